# baseline (speedup 1.0000x reference)
import functools

import jax
import jax.numpy as jnp
from jax import lax
from jax.experimental import pallas as pl
from jax.experimental.pallas import tpu as pltpu
from jax.experimental.pallas import tpu_sc as plsc

B = 16
P = 1024
S1 = 512
S2 = 128
K = 64
NUM_CLASS = 10

SC_NC = 2
SC_NS = 16
SC_NW = SC_NC * SC_NS
SC_L = 16


_FPS_B = B // 2


def _fps_body(px, py, pz, n_pts, n_sample):
    nb = px.shape[0]
    iota_p = lax.broadcasted_iota(jnp.int32, (nb, n_pts), 1)
    iota_s = lax.broadcasted_iota(jnp.int32, (nb, n_sample), 1)

    selx0 = px[:, 0:1]
    sely0 = py[:, 0:1]
    selz0 = pz[:, 0:1]
    dists = (px - selx0) ** 2 + (py - sely0) ** 2 + (pz - selz0) ** 2

    idx_acc = jnp.zeros((nb, n_sample), jnp.int32)
    p1x = jnp.where(iota_s == 0, selx0, 0.0)
    p1y = jnp.where(iota_s == 0, sely0, 0.0)
    p1z = jnp.where(iota_s == 0, selz0, 0.0)

    def body(i, carry):
        dists, idx_acc, p1x, p1y, p1z = carry
        m = jnp.max(dists, axis=1, keepdims=True)
        cand = jnp.where(dists == m, iota_p, n_pts * 2)
        nxt = jnp.min(cand, axis=1, keepdims=True)
        onehot = iota_p == nxt
        selx = jnp.sum(jnp.where(onehot, px, 0.0), axis=1, keepdims=True)
        sely = jnp.sum(jnp.where(onehot, py, 0.0), axis=1, keepdims=True)
        selz = jnp.sum(jnp.where(onehot, pz, 0.0), axis=1, keepdims=True)
        d = (px - selx) ** 2 + (py - sely) ** 2 + (pz - selz) ** 2
        dists = jnp.minimum(dists, d)
        here = iota_s == i
        idx_acc = jnp.where(here, nxt, idx_acc)
        p1x = jnp.where(here, selx, p1x)
        p1y = jnp.where(here, sely, p1y)
        p1z = jnp.where(here, selz, p1z)
        return dists, idx_acc, p1x, p1y, p1z

    carry = (dists, idx_acc, p1x, p1y, p1z)
    carry = lax.fori_loop(1, n_sample, body, carry)
    _, idx_acc, p1x, p1y, p1z = carry
    return idx_acc, p1x, p1y, p1z


def _fps_kernel(px_ref, py_ref, pz_ref,
                idx1_ref, p1x_ref, p1y_ref, p1z_ref,
                idx2_ref, p2x_ref, p2y_ref, p2z_ref):
    px = px_ref[...]
    py = py_ref[...]
    pz = pz_ref[...]
    idx1, p1x, p1y, p1z = _fps_body(px, py, pz, P, S1)
    idx1_ref[...] = idx1
    p1x_ref[...] = p1x
    p1y_ref[...] = p1y
    p1z_ref[...] = p1z
    idx2, p2x, p2y, p2z = _fps_body(p1x, p1y, p1z, S1, S2)
    idx2_ref[...] = idx2
    p2x_ref[...] = p2x
    p2y_ref[...] = p2y
    p2z_ref[...] = p2z


def _run_fps(px, py, pz):
    out_shape = (
        jax.ShapeDtypeStruct((B, S1), jnp.int32),
        jax.ShapeDtypeStruct((B, S1), jnp.float32),
        jax.ShapeDtypeStruct((B, S1), jnp.float32),
        jax.ShapeDtypeStruct((B, S1), jnp.float32),
        jax.ShapeDtypeStruct((B, S2), jnp.int32),
        jax.ShapeDtypeStruct((B, S2), jnp.float32),
        jax.ShapeDtypeStruct((B, S2), jnp.float32),
        jax.ShapeDtypeStruct((B, S2), jnp.float32),
    )
    in_spec = pl.BlockSpec((_FPS_B, P), lambda i: (i, 0))
    s1_spec = pl.BlockSpec((_FPS_B, S1), lambda i: (i, 0))
    s2_spec = pl.BlockSpec((_FPS_B, S2), lambda i: (i, 0))
    return pl.pallas_call(
        _fps_kernel,
        grid=(B // _FPS_B,),
        in_specs=[in_spec] * 3,
        out_specs=(s1_spec,) * 4 + (s2_spec,) * 4,
        out_shape=out_shape,
        compiler_params=pltpu.CompilerParams(
            dimension_semantics=("parallel",)),
    )(px, py, pz)


_ROWS_BLK = 4096


def _pc1_kernel(rel_ref, w1_ref, b1_ref, w2_ref, b2_ref,
                w3_ref, b3_ref, out_ref):
    h = jnp.dot(rel_ref[...], w1_ref[...], preferred_element_type=jnp.float32)
    h = jnp.maximum(h + b1_ref[...], 0.0)
    h = jnp.dot(h, w2_ref[...], preferred_element_type=jnp.float32)
    h = jnp.maximum(h + b2_ref[...], 0.0)
    h = jnp.dot(h, w3_ref[...], preferred_element_type=jnp.float32)
    h = h + b3_ref[...]
    out_ref[...] = jnp.max(h.reshape(_ROWS_BLK // K, K, h.shape[-1]), axis=1)


def _run_pc1(rel, layers):
    (w1, b1), (w2, b2), (w3, b3) = layers
    n = rel.shape[0]
    grid = n // _ROWS_BLK
    qblk = _ROWS_BLK // K
    co = w3.shape[1]
    full = lambda a: pl.BlockSpec(a.shape, lambda i: (0,) * a.ndim)
    return pl.pallas_call(
        _pc1_kernel,
        grid=(grid,),
        in_specs=[
            pl.BlockSpec((_ROWS_BLK, 3), lambda i: (i, 0)),
            full(w1), full(b1.reshape(1, -1)),
            full(w2), full(b2.reshape(1, -1)),
            full(w3), full(b3.reshape(1, -1)),
        ],
        out_specs=pl.BlockSpec((qblk, co), lambda i: (i, 0)),
        out_shape=jax.ShapeDtypeStruct((n // K, co), jnp.float32),
        compiler_params=pltpu.CompilerParams(
            dimension_semantics=("parallel",)),
    )(rel, w1, b1.reshape(1, -1), w2, b2.reshape(1, -1),
      w3, b3.reshape(1, -1))


def _pc2_kernel(xj_ref, rel_ref, w1a_ref, w1b_ref, b1_ref,
                w2_ref, b2_ref, w3_ref, b3_ref, out_ref):
    h = jnp.dot(xj_ref[...], w1a_ref[...], preferred_element_type=jnp.float32)
    h = h + jnp.dot(rel_ref[...], w1b_ref[...],
                    preferred_element_type=jnp.float32)
    h = jnp.maximum(h + b1_ref[...], 0.0)
    h = jnp.dot(h, w2_ref[...], preferred_element_type=jnp.float32)
    h = jnp.maximum(h + b2_ref[...], 0.0)
    h = jnp.dot(h, w3_ref[...], preferred_element_type=jnp.float32)
    h = h + b3_ref[...]
    out_ref[...] = jnp.max(h.reshape(_ROWS_BLK // K, K, h.shape[-1]), axis=1)


def _run_pc2(xj, rel, layers):
    (w1, b1), (w2, b2), (w3, b3) = layers
    ci = xj.shape[1]
    w1a, w1b = w1[:ci], w1[ci:]
    n = xj.shape[0]
    grid = n // _ROWS_BLK
    qblk = _ROWS_BLK // K
    co = w3.shape[1]
    full = lambda a: pl.BlockSpec(a.shape, lambda i: (0,) * a.ndim)
    return pl.pallas_call(
        _pc2_kernel,
        grid=(grid,),
        in_specs=[
            pl.BlockSpec((_ROWS_BLK, ci), lambda i: (i, 0)),
            pl.BlockSpec((_ROWS_BLK, 3), lambda i: (i, 0)),
            full(w1a), full(w1b), full(b1.reshape(1, -1)),
            full(w2), full(b2.reshape(1, -1)),
            full(w3), full(b3.reshape(1, -1)),
        ],
        out_specs=pl.BlockSpec((qblk, co), lambda i: (i, 0)),
        out_shape=jax.ShapeDtypeStruct((n // K, co), jnp.float32),
        compiler_params=pltpu.CompilerParams(
            dimension_semantics=("parallel",)),
    )(xj, rel, w1a, w1b, b1.reshape(1, -1), w2, b2.reshape(1, -1),
      w3, b3.reshape(1, -1))


def _glob_kernel(feat_ref, w1_ref, b1_ref, w2_ref, b2_ref, w3_ref, b3_ref,
                 out_ref):
    h = jnp.dot(feat_ref[...], w1_ref[...], preferred_element_type=jnp.float32)
    h = jnp.maximum(h + b1_ref[...], 0.0)
    h = jnp.dot(h, w2_ref[...], preferred_element_type=jnp.float32)
    h = jnp.maximum(h + b2_ref[...], 0.0)
    h = jnp.dot(h, w3_ref[...], preferred_element_type=jnp.float32)
    h = h + b3_ref[...]
    out_ref[...] = jnp.max(h, axis=0, keepdims=True)[None]


def _run_glob(feat, layers):
    (w1, b1), (w2, b2), (w3, b3) = layers
    ci = feat.shape[1]
    co = w3.shape[1]
    full = lambda a: pl.BlockSpec(a.shape, lambda i: (0,) * a.ndim)
    return pl.pallas_call(
        _glob_kernel,
        grid=(B,),
        in_specs=[
            pl.BlockSpec((S2, ci), lambda i: (i, 0)),
            full(w1), full(b1.reshape(1, -1)),
            full(w2), full(b2.reshape(1, -1)),
            full(w3), full(b3.reshape(1, -1)),
        ],
        out_specs=pl.BlockSpec((1, 1, co), lambda i: (i, 0, 0)),
        out_shape=jax.ShapeDtypeStruct((B, 1, co), jnp.float32),
        compiler_params=pltpu.CompilerParams(
            dimension_semantics=("parallel",)),
    )(feat, w1, b1.reshape(1, -1), w2, b2.reshape(1, -1), w3,
      b3.reshape(1, -1)).reshape(B, co)


def _head_kernel(g_ref, w1_ref, b1_ref, w2_ref, b2_ref, out_ref):
    h = jnp.dot(g_ref[...], w1_ref[...], preferred_element_type=jnp.float32)
    h = jnp.maximum(h + b1_ref[...], 0.0)
    h = jnp.dot(h, w2_ref[...], preferred_element_type=jnp.float32)
    h = h + b2_ref[...]
    m = jnp.max(h, axis=1, keepdims=True)
    e = jnp.exp(h - m)
    out_ref[...] = (h - m) - jnp.log(jnp.sum(e, axis=1, keepdims=True))


def _run_head(g, layers):
    (w1, b1), (w2, b2) = layers
    return pl.pallas_call(
        _head_kernel,
        out_shape=jax.ShapeDtypeStruct((B, NUM_CLASS), jnp.float32),
    )(g, w1, b1.reshape(1, -1), w2, b2.reshape(1, -1))


_BUF = K + 2 * SC_L


def _search_row(pxv, pyv, pzv, bufv, qxs, qys, qzs, selfs, rr, n_chunks,
                iota16):
    trash = K + SC_L + iota16
    for s in range(_BUF // SC_L):
        bufv[pl.ds(s * SC_L, SC_L)] = selfs

    def chunk(c, cnt):
        base = c * SC_L
        dx = pxv[pl.ds(base, SC_L)] - qxs
        dy = pyv[pl.ds(base, SC_L)] - qys
        dz = pzv[pl.ds(base, SC_L)] - qzs
        dsq = dx * dx + dy * dy + dz * dz
        mask = dsq <= rr
        mi = mask.astype(jnp.int32)
        cums = plsc.cumsum(mi)
        slots = jnp.where(mask, cnt + cums - mi, trash)
        plsc.store_scatter(bufv, [slots], iota16 + base)
        return jnp.minimum(cnt + cums[SC_L - 1], K)

    lax.fori_loop(0, n_chunks, chunk, 0)


_QW1 = S1 // 2


def _rs1_kernel(px_hbm, py_hbm, pz_hbm, qx_hbm, qy_hbm, qz_hbm, self_hbm,
                rx_hbm, ry_hbm, rz_hbm,
                pxv, pyv, pzv, qxv, qyv, qzv, selfv, bufv, rxv, ryv, rzv):
    wid = lax.axis_index("s") * SC_NC + lax.axis_index("c")
    b = wid // 2
    h = wid % 2
    pltpu.sync_copy(px_hbm.at[b], pxv)
    pltpu.sync_copy(py_hbm.at[b], pyv)
    pltpu.sync_copy(pz_hbm.at[b], pzv)
    q0 = h * _QW1
    pltpu.sync_copy(qx_hbm.at[b, pl.ds(q0, _QW1)], qxv)
    pltpu.sync_copy(qy_hbm.at[b, pl.ds(q0, _QW1)], qyv)
    pltpu.sync_copy(qz_hbm.at[b, pl.ds(q0, _QW1)], qzv)
    pltpu.sync_copy(self_hbm.at[b, pl.ds(q0, _QW1)], selfv)
    iota16 = lax.broadcasted_iota(jnp.int32, (SC_L,), 0)
    rr = jnp.float32(0.2 * 0.2)

    def qchunk(qb, _):
        qx16 = qxv[pl.ds(qb * SC_L, SC_L)]
        qy16 = qyv[pl.ds(qb * SC_L, SC_L)]
        qz16 = qzv[pl.ds(qb * SC_L, SC_L)]
        self16 = selfv[pl.ds(qb * SC_L, SC_L)]
        for j in range(SC_L):
            qi = qb * SC_L + j
            qxs = jnp.full((SC_L,), qx16[j], jnp.float32)
            qys = jnp.full((SC_L,), qy16[j], jnp.float32)
            qzs = jnp.full((SC_L,), qz16[j], jnp.float32)
            selfs = jnp.full((SC_L,), self16[j], jnp.int32)
            _search_row(pxv, pyv, pzv, bufv, qxs, qys, qzs, selfs, rr,
                        P // SC_L, iota16)
            for s in range(K // SC_L):
                idxv = bufv[pl.ds(s * SC_L, SC_L)]
                rxv[qi, pl.ds(s * SC_L, SC_L)] = (
                    plsc.load_gather(pxv, [idxv]) - qxs)
                ryv[qi, pl.ds(s * SC_L, SC_L)] = (
                    plsc.load_gather(pyv, [idxv]) - qys)
                rzv[qi, pl.ds(s * SC_L, SC_L)] = (
                    plsc.load_gather(pzv, [idxv]) - qzs)
        return 0

    lax.fori_loop(0, _QW1 // SC_L, qchunk, 0)
    pltpu.sync_copy(rxv, rx_hbm.at[b, pl.ds(q0, _QW1)])
    pltpu.sync_copy(ryv, ry_hbm.at[b, pl.ds(q0, _QW1)])
    pltpu.sync_copy(rzv, rz_hbm.at[b, pl.ds(q0, _QW1)])


def _run_rs1(px, py, pz, qx, qy, qz, self_idx):
    mesh = plsc.VectorSubcoreMesh(core_axis_name="c", subcore_axis_name="s",
                                  num_cores=SC_NC, num_subcores=SC_NS)
    f32 = jnp.float32
    out_type = tuple(jax.ShapeDtypeStruct((B, S1, K), f32) for _ in range(3))
    fn = pl.kernel(
        _rs1_kernel,
        out_type=out_type,
        mesh=mesh,
        scratch_types=[
            pltpu.VMEM((P,), f32), pltpu.VMEM((P,), f32),
            pltpu.VMEM((P,), f32),
            pltpu.VMEM((_QW1,), f32), pltpu.VMEM((_QW1,), f32),
            pltpu.VMEM((_QW1,), f32),
            pltpu.VMEM((_QW1,), jnp.int32),
            pltpu.VMEM((_BUF,), jnp.int32),
            pltpu.VMEM((_QW1, K), f32), pltpu.VMEM((_QW1, K), f32),
            pltpu.VMEM((_QW1, K), f32),
        ],
        compiler_params=pltpu.CompilerParams(needs_layout_passes=False),
    )
    return fn(px, py, pz, qx, qy, qz, self_idx)


_QW2 = S2 // 2
_GRP = 8


def _rs2_kernel(px_hbm, py_hbm, pz_hbm, qx_hbm, qy_hbm, qz_hbm, self_hbm,
                x1_hbm,
                rx_hbm, ry_hbm, rz_hbm, xj_hbm,
                pxv, pyv, pzv, qxv, qyv, qzv, selfv, bufv, rxv, ryv, rzv,
                idxg, rows_v, sem):
    wid = lax.axis_index("s") * SC_NC + lax.axis_index("c")
    b = wid // 2
    h = wid % 2
    pltpu.sync_copy(px_hbm.at[b], pxv)
    pltpu.sync_copy(py_hbm.at[b], pyv)
    pltpu.sync_copy(pz_hbm.at[b], pzv)
    q0 = h * _QW2
    pltpu.sync_copy(qx_hbm.at[b, pl.ds(q0, _QW2)], qxv)
    pltpu.sync_copy(qy_hbm.at[b, pl.ds(q0, _QW2)], qyv)
    pltpu.sync_copy(qz_hbm.at[b, pl.ds(q0, _QW2)], qzv)
    pltpu.sync_copy(self_hbm.at[b, pl.ds(q0, _QW2)], selfv)
    iota16 = lax.broadcasted_iota(jnp.int32, (SC_L,), 0)
    rr = jnp.float32(0.4 * 0.4)
    row_base = jnp.int32(b * S2 + q0)

    def qchunk(qb, _):
        qx16 = qxv[pl.ds(qb * SC_L, SC_L)]
        qy16 = qyv[pl.ds(qb * SC_L, SC_L)]
        qz16 = qzv[pl.ds(qb * SC_L, SC_L)]
        self16 = selfv[pl.ds(qb * SC_L, SC_L)]
        for half in range(SC_L // _GRP):
            for j in range(_GRP):
                lane = half * _GRP + j
                qi = qb * SC_L + lane
                qxs = jnp.full((SC_L,), qx16[lane], jnp.float32)
                qys = jnp.full((SC_L,), qy16[lane], jnp.float32)
                qzs = jnp.full((SC_L,), qz16[lane], jnp.float32)
                selfs = jnp.full((SC_L,), self16[lane], jnp.int32)
                _search_row(pxv, pyv, pzv, bufv, qxs, qys, qzs, selfs, rr,
                            S1 // SC_L, iota16)
                for s in range(K // SC_L):
                    idxv = bufv[pl.ds(s * SC_L, SC_L)]
                    rxv[qi, pl.ds(s * SC_L, SC_L)] = (
                        plsc.load_gather(pxv, [idxv]) - qxs)
                    ryv[qi, pl.ds(s * SC_L, SC_L)] = (
                        plsc.load_gather(pyv, [idxv]) - qys)
                    rzv[qi, pl.ds(s * SC_L, SC_L)] = (
                        plsc.load_gather(pzv, [idxv]) - qzs)
                    idxg[pl.ds(j * K + s * SC_L, SC_L)] = idxv + b * S1
            pltpu.async_copy(x1_hbm.at[idxg], rows_v, sem).wait()
            row0 = row_base + qb * SC_L + half * _GRP
            pltpu.sync_copy(rows_v, xj_hbm.at[pl.ds(row0 * K, _GRP * K)])
        return 0

    lax.fori_loop(0, _QW2 // SC_L, qchunk, 0)
    pltpu.sync_copy(rxv, rx_hbm.at[b, pl.ds(q0, _QW2)])
    pltpu.sync_copy(ryv, ry_hbm.at[b, pl.ds(q0, _QW2)])
    pltpu.sync_copy(rzv, rz_hbm.at[b, pl.ds(q0, _QW2)])


def _run_rs2(px, py, pz, qx, qy, qz, self_idx, x1):
    mesh = plsc.VectorSubcoreMesh(core_axis_name="c", subcore_axis_name="s",
                                  num_cores=SC_NC, num_subcores=SC_NS)
    f32 = jnp.float32
    out_type = (
        jax.ShapeDtypeStruct((B, S2, K), f32),
        jax.ShapeDtypeStruct((B, S2, K), f32),
        jax.ShapeDtypeStruct((B, S2, K), f32),
        jax.ShapeDtypeStruct((B * S2 * K, 128), f32),
    )
    fn = pl.kernel(
        _rs2_kernel,
        out_type=out_type,
        mesh=mesh,
        scratch_types=[
            pltpu.VMEM((S1,), f32), pltpu.VMEM((S1,), f32),
            pltpu.VMEM((S1,), f32),
            pltpu.VMEM((_QW2,), f32), pltpu.VMEM((_QW2,), f32),
            pltpu.VMEM((_QW2,), f32),
            pltpu.VMEM((_QW2,), jnp.int32),
            pltpu.VMEM((_BUF,), jnp.int32),
            pltpu.VMEM((_QW2, K), f32), pltpu.VMEM((_QW2, K), f32),
            pltpu.VMEM((_QW2, K), f32),
            pltpu.VMEM((_GRP * K,), jnp.int32),
            pltpu.VMEM((_GRP * K, 128), f32),
            pltpu.SemaphoreType.DMA,
        ],
        compiler_params=pltpu.CompilerParams(needs_layout_passes=False),
    )
    return fn(px, py, pz, qx, qy, qz, self_idx, x1)


def kernel(pos, batch, params):
    del batch
    pos = pos.reshape(B, P, 3)
    px, py, pz = pos[:, :, 0], pos[:, :, 1], pos[:, :, 2]
    (idx1, p1x, p1y, p1z, idx2, p2x, p2y, p2z) = _run_fps(px, py, pz)

    rx1, ry1, rz1 = _run_rs1(px, py, pz, p1x, p1y, p1z, idx1)
    rel1 = jnp.stack(
        [rx1.reshape(-1), ry1.reshape(-1), rz1.reshape(-1)], axis=-1)
    x1 = _run_pc1(rel1, params['sa1'])

    rx2, ry2, rz2, xj2 = _run_rs2(p1x, p1y, p1z, p2x, p2y, p2z, idx2, x1)
    rel2 = jnp.stack(
        [rx2.reshape(-1), ry2.reshape(-1), rz2.reshape(-1)], axis=-1)
    x2 = _run_pc2(xj2, rel2, params['sa2'])

    pos2 = jnp.stack([p2x, p2y, p2z], axis=-1)
    feat = jnp.concatenate([x2, pos2.reshape(B * S2, 3)], axis=-1)
    g = _run_glob(feat, params['sa3'])
    return _run_head(g, params['head'])


# speedup vs baseline: 1.3754x; 1.3754x over previous
import functools

import jax
import jax.numpy as jnp
from jax import lax
from jax.experimental import pallas as pl
from jax.experimental.pallas import tpu as pltpu
from jax.experimental.pallas import tpu_sc as plsc

B = 16
P = 1024
S1 = 512
S2 = 128
K = 64
NUM_CLASS = 10

SC_NC = 2
SC_NS = 16
SC_NW = SC_NC * SC_NS
SC_L = 16


_FPS_B = B // 2


def _fps_body(px, py, pz, n_pts, n_sample):
    nb = px.shape[0]
    iota_p = lax.broadcasted_iota(jnp.int32, (nb, n_pts), 1)
    iota_s = lax.broadcasted_iota(jnp.int32, (nb, n_sample), 1)

    selx0 = px[:, 0:1]
    sely0 = py[:, 0:1]
    selz0 = pz[:, 0:1]
    dists = (px - selx0) ** 2 + (py - sely0) ** 2 + (pz - selz0) ** 2

    idx_acc = jnp.zeros((nb, n_sample), jnp.int32)
    p1x = jnp.where(iota_s == 0, selx0, 0.0)
    p1y = jnp.where(iota_s == 0, sely0, 0.0)
    p1z = jnp.where(iota_s == 0, selz0, 0.0)

    def body(i, carry):
        dists, idx_acc, p1x, p1y, p1z = carry
        m = jnp.max(dists, axis=1, keepdims=True)
        cand = jnp.where(dists == m, iota_p, n_pts * 2)
        nxt = jnp.min(cand, axis=1, keepdims=True)
        onehot = iota_p == nxt
        selx = jnp.sum(jnp.where(onehot, px, 0.0), axis=1, keepdims=True)
        sely = jnp.sum(jnp.where(onehot, py, 0.0), axis=1, keepdims=True)
        selz = jnp.sum(jnp.where(onehot, pz, 0.0), axis=1, keepdims=True)
        d = (px - selx) ** 2 + (py - sely) ** 2 + (pz - selz) ** 2
        dists = jnp.minimum(dists, d)
        here = iota_s == i
        idx_acc = jnp.where(here, nxt, idx_acc)
        p1x = jnp.where(here, selx, p1x)
        p1y = jnp.where(here, sely, p1y)
        p1z = jnp.where(here, selz, p1z)
        return dists, idx_acc, p1x, p1y, p1z

    carry = (dists, idx_acc, p1x, p1y, p1z)
    carry = lax.fori_loop(1, n_sample, body, carry)
    _, idx_acc, p1x, p1y, p1z = carry
    return idx_acc, p1x, p1y, p1z


def _fps_kernel(px_ref, py_ref, pz_ref,
                idx1_ref, p1x_ref, p1y_ref, p1z_ref,
                idx2_ref, p2x_ref, p2y_ref, p2z_ref):
    px = px_ref[...]
    py = py_ref[...]
    pz = pz_ref[...]
    idx1, p1x, p1y, p1z = _fps_body(px, py, pz, P, S1)
    idx1_ref[...] = idx1
    p1x_ref[...] = p1x
    p1y_ref[...] = p1y
    p1z_ref[...] = p1z
    idx2, p2x, p2y, p2z = _fps_body(p1x, p1y, p1z, S1, S2)
    idx2_ref[...] = idx2
    p2x_ref[...] = p2x
    p2y_ref[...] = p2y
    p2z_ref[...] = p2z


def _run_fps(px, py, pz):
    out_shape = (
        jax.ShapeDtypeStruct((B, S1), jnp.int32),
        jax.ShapeDtypeStruct((B, S1), jnp.float32),
        jax.ShapeDtypeStruct((B, S1), jnp.float32),
        jax.ShapeDtypeStruct((B, S1), jnp.float32),
        jax.ShapeDtypeStruct((B, S2), jnp.int32),
        jax.ShapeDtypeStruct((B, S2), jnp.float32),
        jax.ShapeDtypeStruct((B, S2), jnp.float32),
        jax.ShapeDtypeStruct((B, S2), jnp.float32),
    )
    return pl.pallas_call(_fps_kernel, out_shape=out_shape)(px, py, pz)


_ROWS_BLK = 4096


def _pc1_kernel(rel_ref, w1_ref, b1_ref, w2_ref, b2_ref,
                w3_ref, b3_ref, out_ref):
    h = jnp.dot(rel_ref[...], w1_ref[...], preferred_element_type=jnp.float32)
    h = jnp.maximum(h + b1_ref[...], 0.0)
    h = jnp.dot(h, w2_ref[...], preferred_element_type=jnp.float32)
    h = jnp.maximum(h + b2_ref[...], 0.0)
    h = jnp.dot(h, w3_ref[...], preferred_element_type=jnp.float32)
    h = h + b3_ref[...]
    out_ref[...] = jnp.max(h.reshape(_ROWS_BLK // K, K, h.shape[-1]), axis=1)


def _run_pc1(rel, layers):
    (w1, b1), (w2, b2), (w3, b3) = layers
    n = rel.shape[0]
    grid = n // _ROWS_BLK
    qblk = _ROWS_BLK // K
    co = w3.shape[1]
    full = lambda a: pl.BlockSpec(a.shape, lambda i: (0,) * a.ndim)
    return pl.pallas_call(
        _pc1_kernel,
        grid=(grid,),
        in_specs=[
            pl.BlockSpec((_ROWS_BLK, 3), lambda i: (i, 0)),
            full(w1), full(b1.reshape(1, -1)),
            full(w2), full(b2.reshape(1, -1)),
            full(w3), full(b3.reshape(1, -1)),
        ],
        out_specs=pl.BlockSpec((qblk, co), lambda i: (i, 0)),
        out_shape=jax.ShapeDtypeStruct((n // K, co), jnp.float32),
    )(rel, w1, b1.reshape(1, -1), w2, b2.reshape(1, -1),
      w3, b3.reshape(1, -1))


def _pc2_kernel(xj_ref, rel_ref, w1a_ref, w1b_ref, b1_ref,
                w2_ref, b2_ref, w3_ref, b3_ref, out_ref):
    h = jnp.dot(xj_ref[...], w1a_ref[...], preferred_element_type=jnp.float32)
    h = h + jnp.dot(rel_ref[...], w1b_ref[...],
                    preferred_element_type=jnp.float32)
    h = jnp.maximum(h + b1_ref[...], 0.0)
    h = jnp.dot(h, w2_ref[...], preferred_element_type=jnp.float32)
    h = jnp.maximum(h + b2_ref[...], 0.0)
    h = jnp.dot(h, w3_ref[...], preferred_element_type=jnp.float32)
    h = h + b3_ref[...]
    out_ref[...] = jnp.max(h.reshape(_ROWS_BLK // K, K, h.shape[-1]), axis=1)


def _run_pc2(xj, rel, layers):
    (w1, b1), (w2, b2), (w3, b3) = layers
    ci = xj.shape[1]
    w1a, w1b = w1[:ci], w1[ci:]
    n = xj.shape[0]
    grid = n // _ROWS_BLK
    qblk = _ROWS_BLK // K
    co = w3.shape[1]
    full = lambda a: pl.BlockSpec(a.shape, lambda i: (0,) * a.ndim)
    return pl.pallas_call(
        _pc2_kernel,
        grid=(grid,),
        in_specs=[
            pl.BlockSpec((_ROWS_BLK, ci), lambda i: (i, 0)),
            pl.BlockSpec((_ROWS_BLK, 3), lambda i: (i, 0)),
            full(w1a), full(w1b), full(b1.reshape(1, -1)),
            full(w2), full(b2.reshape(1, -1)),
            full(w3), full(b3.reshape(1, -1)),
        ],
        out_specs=pl.BlockSpec((qblk, co), lambda i: (i, 0)),
        out_shape=jax.ShapeDtypeStruct((n // K, co), jnp.float32),
    )(xj, rel, w1a, w1b, b1.reshape(1, -1), w2, b2.reshape(1, -1),
      w3, b3.reshape(1, -1))


def _glob_kernel(feat_ref, w1_ref, b1_ref, w2_ref, b2_ref, w3_ref, b3_ref,
                 out_ref):
    h = jnp.dot(feat_ref[...], w1_ref[...], preferred_element_type=jnp.float32)
    h = jnp.maximum(h + b1_ref[...], 0.0)
    h = jnp.dot(h, w2_ref[...], preferred_element_type=jnp.float32)
    h = jnp.maximum(h + b2_ref[...], 0.0)
    h = jnp.dot(h, w3_ref[...], preferred_element_type=jnp.float32)
    h = h + b3_ref[...]
    out_ref[...] = jnp.max(h, axis=0, keepdims=True)[None]


def _run_glob(feat, layers):
    (w1, b1), (w2, b2), (w3, b3) = layers
    ci = feat.shape[1]
    co = w3.shape[1]
    full = lambda a: pl.BlockSpec(a.shape, lambda i: (0,) * a.ndim)
    return pl.pallas_call(
        _glob_kernel,
        grid=(B,),
        in_specs=[
            pl.BlockSpec((S2, ci), lambda i: (i, 0)),
            full(w1), full(b1.reshape(1, -1)),
            full(w2), full(b2.reshape(1, -1)),
            full(w3), full(b3.reshape(1, -1)),
        ],
        out_specs=pl.BlockSpec((1, 1, co), lambda i: (i, 0, 0)),
        out_shape=jax.ShapeDtypeStruct((B, 1, co), jnp.float32),
    )(feat, w1, b1.reshape(1, -1), w2, b2.reshape(1, -1), w3,
      b3.reshape(1, -1)).reshape(B, co)


def _head_kernel(g_ref, w1_ref, b1_ref, w2_ref, b2_ref, out_ref):
    h = jnp.dot(g_ref[...], w1_ref[...], preferred_element_type=jnp.float32)
    h = jnp.maximum(h + b1_ref[...], 0.0)
    h = jnp.dot(h, w2_ref[...], preferred_element_type=jnp.float32)
    h = h + b2_ref[...]
    m = jnp.max(h, axis=1, keepdims=True)
    e = jnp.exp(h - m)
    out_ref[...] = (h - m) - jnp.log(jnp.sum(e, axis=1, keepdims=True))


def _run_head(g, layers):
    (w1, b1), (w2, b2) = layers
    return pl.pallas_call(
        _head_kernel,
        out_shape=jax.ShapeDtypeStruct((B, NUM_CLASS), jnp.float32),
    )(g, w1, b1.reshape(1, -1), w2, b2.reshape(1, -1))


_BUF = K + 2 * SC_L
_NBKT = 16


def _bucket_of(z16):
    b = ((z16 + 1.0) * (_NBKT / 2.0)).astype(jnp.int32)
    return jnp.clip(b, 0, _NBKT - 1)


def _build_zbuckets(pxv, pyv, pzv, ppxv, ppyv, ppzv, ppiv, startsv, n_pts,
                    iota16):
    n_chunks = n_pts // SC_L
    cnt = jnp.int32(0)
    for k in range(_NBKT):
        plsc.store_scatter(startsv, [jnp.full((SC_L,), k, jnp.int32)],
                           jnp.full((SC_L,), cnt, jnp.int32))

        def chunk(c, cnt, k=k):
            z = pzv[pl.ds(c * SC_L, SC_L)]
            mask = _bucket_of(z) == k
            mi = mask.astype(jnp.int32)
            cums = plsc.cumsum(mi)
            slots = jnp.where(mask, cnt + cums - mi, n_pts + iota16)
            plsc.store_scatter(ppiv, [slots], iota16 + c * SC_L)
            return cnt + cums[SC_L - 1]

        cnt = lax.fori_loop(0, n_chunks, chunk, cnt)
    plsc.store_scatter(startsv, [jnp.full((SC_L,), _NBKT, jnp.int32)],
                       jnp.full((SC_L,), n_pts, jnp.int32))

    def fill(c, _):
        idxv = ppiv[pl.ds(c * SC_L, SC_L)]
        ppxv[pl.ds(c * SC_L, SC_L)] = plsc.load_gather(pxv, [idxv])
        ppyv[pl.ds(c * SC_L, SC_L)] = plsc.load_gather(pyv, [idxv])
        ppzv[pl.ds(c * SC_L, SC_L)] = plsc.load_gather(pzv, [idxv])
        return 0

    lax.fori_loop(0, n_chunks, fill, 0)


def _search_row(ppxv, ppyv, ppzv, ppiv, bufv, qxs, qys, qzs, selfs, rr,
                c0, c1, iota16):
    trash = K + SC_L + iota16
    for s in range(_BUF // SC_L):
        bufv[pl.ds(s * SC_L, SC_L)] = selfs

    def chunk(c, cnt):
        base = c * SC_L
        dx = ppxv[pl.ds(base, SC_L)] - qxs
        dy = ppyv[pl.ds(base, SC_L)] - qys
        dz = ppzv[pl.ds(base, SC_L)] - qzs
        dsq = dx * dx + dy * dy + dz * dz
        mask = dsq <= rr
        mi = mask.astype(jnp.int32)
        cums = plsc.cumsum(mi)
        slots = jnp.where(mask, cnt + cums - mi, trash)
        plsc.store_scatter(bufv, [slots], ppiv[pl.ds(base, SC_L)])
        return jnp.minimum(cnt + cums[SC_L - 1], K)

    lax.fori_loop(c0, c1, chunk, 0)


_QW1 = S1 // 2


def _rs1_kernel(px_hbm, py_hbm, pz_hbm, qx_hbm, qy_hbm, qz_hbm, self_hbm,
                rx_hbm, ry_hbm, rz_hbm,
                pxv, pyv, pzv, qxv, qyv, qzv, selfv, bufv, rxv, ryv, rzv,
                ppxv, ppyv, ppzv, ppiv, startsv):
    wid = lax.axis_index("s") * SC_NC + lax.axis_index("c")
    b = wid // 2
    h = wid % 2
    pltpu.sync_copy(px_hbm.at[b], pxv)
    pltpu.sync_copy(py_hbm.at[b], pyv)
    pltpu.sync_copy(pz_hbm.at[b], pzv)
    q0 = h * _QW1
    pltpu.sync_copy(qx_hbm.at[b, pl.ds(q0, _QW1)], qxv)
    pltpu.sync_copy(qy_hbm.at[b, pl.ds(q0, _QW1)], qyv)
    pltpu.sync_copy(qz_hbm.at[b, pl.ds(q0, _QW1)], qzv)
    pltpu.sync_copy(self_hbm.at[b, pl.ds(q0, _QW1)], selfv)
    iota16 = lax.broadcasted_iota(jnp.int32, (SC_L,), 0)
    r = 0.2
    rr = jnp.float32(r * r)
    _build_zbuckets(pxv, pyv, pzv, ppxv, ppyv, ppzv, ppiv, startsv, P,
                    iota16)

    def qchunk(qb, _):
        qx16 = qxv[pl.ds(qb * SC_L, SC_L)]
        qy16 = qyv[pl.ds(qb * SC_L, SC_L)]
        qz16 = qzv[pl.ds(qb * SC_L, SC_L)]
        self16 = selfv[pl.ds(qb * SC_L, SC_L)]
        s16 = plsc.load_gather(startsv, [_bucket_of(qz16 - r)])
        e16 = plsc.load_gather(startsv, [_bucket_of(qz16 + r) + 1])
        for j in range(SC_L):
            qi = qb * SC_L + j
            qxs = jnp.full((SC_L,), qx16[j], jnp.float32)
            qys = jnp.full((SC_L,), qy16[j], jnp.float32)
            qzs = jnp.full((SC_L,), qz16[j], jnp.float32)
            selfs = jnp.full((SC_L,), self16[j], jnp.int32)
            c0 = lax.shift_right_logical(s16[j], 4)
            c1 = lax.shift_right_logical(e16[j] + (SC_L - 1), 4)
            _search_row(ppxv, ppyv, ppzv, ppiv, bufv, qxs, qys, qzs, selfs,
                        rr, c0, c1, iota16)
            for s in range(K // SC_L):
                idxv = bufv[pl.ds(s * SC_L, SC_L)]
                rxv[qi, pl.ds(s * SC_L, SC_L)] = (
                    plsc.load_gather(pxv, [idxv]) - qxs)
                ryv[qi, pl.ds(s * SC_L, SC_L)] = (
                    plsc.load_gather(pyv, [idxv]) - qys)
                rzv[qi, pl.ds(s * SC_L, SC_L)] = (
                    plsc.load_gather(pzv, [idxv]) - qzs)
        return 0

    lax.fori_loop(0, _QW1 // SC_L, qchunk, 0)
    pltpu.sync_copy(rxv, rx_hbm.at[b, pl.ds(q0, _QW1)])
    pltpu.sync_copy(ryv, ry_hbm.at[b, pl.ds(q0, _QW1)])
    pltpu.sync_copy(rzv, rz_hbm.at[b, pl.ds(q0, _QW1)])


def _run_rs1(px, py, pz, qx, qy, qz, self_idx):
    mesh = plsc.VectorSubcoreMesh(core_axis_name="c", subcore_axis_name="s",
                                  num_cores=SC_NC, num_subcores=SC_NS)
    f32 = jnp.float32
    out_type = tuple(jax.ShapeDtypeStruct((B, S1, K), f32) for _ in range(3))
    fn = pl.kernel(
        _rs1_kernel,
        out_type=out_type,
        mesh=mesh,
        scratch_types=[
            pltpu.VMEM((P,), f32), pltpu.VMEM((P,), f32),
            pltpu.VMEM((P,), f32),
            pltpu.VMEM((_QW1,), f32), pltpu.VMEM((_QW1,), f32),
            pltpu.VMEM((_QW1,), f32),
            pltpu.VMEM((_QW1,), jnp.int32),
            pltpu.VMEM((_BUF,), jnp.int32),
            pltpu.VMEM((_QW1, K), f32), pltpu.VMEM((_QW1, K), f32),
            pltpu.VMEM((_QW1, K), f32),
            pltpu.VMEM((P,), f32), pltpu.VMEM((P,), f32),
            pltpu.VMEM((P,), f32),
            pltpu.VMEM((P + SC_L,), jnp.int32),
            pltpu.VMEM((2 * SC_L,), jnp.int32),
        ],
        compiler_params=pltpu.CompilerParams(needs_layout_passes=False),
    )
    return fn(px, py, pz, qx, qy, qz, self_idx)


_QW2 = S2 // 2
_GRP = 8


def _rs2_kernel(px_hbm, py_hbm, pz_hbm, qx_hbm, qy_hbm, qz_hbm, self_hbm,
                x1_hbm,
                rx_hbm, ry_hbm, rz_hbm, xj_hbm,
                pxv, pyv, pzv, qxv, qyv, qzv, selfv, bufv, rxv, ryv, rzv,
                idxg, rows_v, sem, ppxv, ppyv, ppzv, ppiv, startsv):
    wid = lax.axis_index("s") * SC_NC + lax.axis_index("c")
    b = wid // 2
    h = wid % 2
    pltpu.sync_copy(px_hbm.at[b], pxv)
    pltpu.sync_copy(py_hbm.at[b], pyv)
    pltpu.sync_copy(pz_hbm.at[b], pzv)
    q0 = h * _QW2
    pltpu.sync_copy(qx_hbm.at[b, pl.ds(q0, _QW2)], qxv)
    pltpu.sync_copy(qy_hbm.at[b, pl.ds(q0, _QW2)], qyv)
    pltpu.sync_copy(qz_hbm.at[b, pl.ds(q0, _QW2)], qzv)
    pltpu.sync_copy(self_hbm.at[b, pl.ds(q0, _QW2)], selfv)
    iota16 = lax.broadcasted_iota(jnp.int32, (SC_L,), 0)
    r = 0.4
    rr = jnp.float32(r * r)
    row_base = jnp.int32(b * S2 + q0)
    _build_zbuckets(pxv, pyv, pzv, ppxv, ppyv, ppzv, ppiv, startsv, S1,
                    iota16)

    def qchunk(qb, _):
        qx16 = qxv[pl.ds(qb * SC_L, SC_L)]
        qy16 = qyv[pl.ds(qb * SC_L, SC_L)]
        qz16 = qzv[pl.ds(qb * SC_L, SC_L)]
        self16 = selfv[pl.ds(qb * SC_L, SC_L)]
        s16 = plsc.load_gather(startsv, [_bucket_of(qz16 - r)])
        e16 = plsc.load_gather(startsv, [_bucket_of(qz16 + r) + 1])
        for half in range(SC_L // _GRP):
            for j in range(_GRP):
                lane = half * _GRP + j
                qi = qb * SC_L + lane
                qxs = jnp.full((SC_L,), qx16[lane], jnp.float32)
                qys = jnp.full((SC_L,), qy16[lane], jnp.float32)
                qzs = jnp.full((SC_L,), qz16[lane], jnp.float32)
                selfs = jnp.full((SC_L,), self16[lane], jnp.int32)
                c0 = lax.shift_right_logical(s16[lane], 4)
                c1 = lax.shift_right_logical(e16[lane] + (SC_L - 1), 4)
                _search_row(ppxv, ppyv, ppzv, ppiv, bufv, qxs, qys, qzs,
                            selfs, rr, c0, c1, iota16)
                for s in range(K // SC_L):
                    idxv = bufv[pl.ds(s * SC_L, SC_L)]
                    rxv[qi, pl.ds(s * SC_L, SC_L)] = (
                        plsc.load_gather(pxv, [idxv]) - qxs)
                    ryv[qi, pl.ds(s * SC_L, SC_L)] = (
                        plsc.load_gather(pyv, [idxv]) - qys)
                    rzv[qi, pl.ds(s * SC_L, SC_L)] = (
                        plsc.load_gather(pzv, [idxv]) - qzs)
                    idxg[pl.ds(j * K + s * SC_L, SC_L)] = idxv + b * S1
            pltpu.async_copy(x1_hbm.at[idxg], rows_v, sem).wait()
            row0 = row_base + qb * SC_L + half * _GRP
            pltpu.sync_copy(rows_v, xj_hbm.at[pl.ds(row0 * K, _GRP * K)])
        return 0

    lax.fori_loop(0, _QW2 // SC_L, qchunk, 0)
    pltpu.sync_copy(rxv, rx_hbm.at[b, pl.ds(q0, _QW2)])
    pltpu.sync_copy(ryv, ry_hbm.at[b, pl.ds(q0, _QW2)])
    pltpu.sync_copy(rzv, rz_hbm.at[b, pl.ds(q0, _QW2)])


def _run_rs2(px, py, pz, qx, qy, qz, self_idx, x1):
    mesh = plsc.VectorSubcoreMesh(core_axis_name="c", subcore_axis_name="s",
                                  num_cores=SC_NC, num_subcores=SC_NS)
    f32 = jnp.float32
    out_type = (
        jax.ShapeDtypeStruct((B, S2, K), f32),
        jax.ShapeDtypeStruct((B, S2, K), f32),
        jax.ShapeDtypeStruct((B, S2, K), f32),
        jax.ShapeDtypeStruct((B * S2 * K, 128), f32),
    )
    fn = pl.kernel(
        _rs2_kernel,
        out_type=out_type,
        mesh=mesh,
        scratch_types=[
            pltpu.VMEM((S1,), f32), pltpu.VMEM((S1,), f32),
            pltpu.VMEM((S1,), f32),
            pltpu.VMEM((_QW2,), f32), pltpu.VMEM((_QW2,), f32),
            pltpu.VMEM((_QW2,), f32),
            pltpu.VMEM((_QW2,), jnp.int32),
            pltpu.VMEM((_BUF,), jnp.int32),
            pltpu.VMEM((_QW2, K), f32), pltpu.VMEM((_QW2, K), f32),
            pltpu.VMEM((_QW2, K), f32),
            pltpu.VMEM((_GRP * K,), jnp.int32),
            pltpu.VMEM((_GRP * K, 128), f32),
            pltpu.SemaphoreType.DMA,
            pltpu.VMEM((S1,), f32), pltpu.VMEM((S1,), f32),
            pltpu.VMEM((S1,), f32),
            pltpu.VMEM((S1 + SC_L,), jnp.int32),
            pltpu.VMEM((2 * SC_L,), jnp.int32),
        ],
        compiler_params=pltpu.CompilerParams(needs_layout_passes=False),
    )
    return fn(px, py, pz, qx, qy, qz, self_idx, x1)


def kernel(pos, batch, params):
    del batch
    pos = pos.reshape(B, P, 3)
    px, py, pz = pos[:, :, 0], pos[:, :, 1], pos[:, :, 2]
    (idx1, p1x, p1y, p1z, idx2, p2x, p2y, p2z) = _run_fps(px, py, pz)

    rx1, ry1, rz1 = _run_rs1(px, py, pz, p1x, p1y, p1z, idx1)
    rel1 = jnp.stack(
        [rx1.reshape(-1), ry1.reshape(-1), rz1.reshape(-1)], axis=-1)
    x1 = _run_pc1(rel1, params['sa1'])

    rx2, ry2, rz2, xj2 = _run_rs2(p1x, p1y, p1z, p2x, p2y, p2z, idx2, x1)
    rel2 = jnp.stack(
        [rx2.reshape(-1), ry2.reshape(-1), rz2.reshape(-1)], axis=-1)
    x2 = _run_pc2(xj2, rel2, params['sa2'])

    pos2 = jnp.stack([p2x, p2y, p2z], axis=-1)
    feat = jnp.concatenate([x2, pos2.reshape(B * S2, 3)], axis=-1)
    g = _run_glob(feat, params['sa3'])
    return _run_head(g, params['head'])


# speedup vs baseline: 1.4020x; 1.0194x over previous
import functools

import jax
import jax.numpy as jnp
from jax import lax
from jax.experimental import pallas as pl
from jax.experimental.pallas import tpu as pltpu
from jax.experimental.pallas import tpu_sc as plsc

B = 16
P = 1024
S1 = 512
S2 = 128
K = 64
NUM_CLASS = 10

SC_NC = 2
SC_NS = 16
SC_NW = SC_NC * SC_NS
SC_L = 16


_FPS_B = B // 2


def _fps_body(px, py, pz, n_pts, n_sample):
    nb = px.shape[0]
    iota_p = lax.broadcasted_iota(jnp.int32, (nb, n_pts), 1)
    iota_s = lax.broadcasted_iota(jnp.int32, (nb, n_sample), 1)

    selx0 = px[:, 0:1]
    sely0 = py[:, 0:1]
    selz0 = pz[:, 0:1]
    dists = (px - selx0) ** 2 + (py - sely0) ** 2 + (pz - selz0) ** 2

    idx_acc = jnp.zeros((nb, n_sample), jnp.int32)
    p1x = jnp.where(iota_s == 0, selx0, 0.0)
    p1y = jnp.where(iota_s == 0, sely0, 0.0)
    p1z = jnp.where(iota_s == 0, selz0, 0.0)

    def body(i, carry):
        dists, idx_acc, p1x, p1y, p1z = carry
        m = jnp.max(dists, axis=1, keepdims=True)
        cand = jnp.where(dists == m, iota_p, n_pts * 2)
        nxt = jnp.min(cand, axis=1, keepdims=True)
        onehot = iota_p == nxt
        selx = jnp.sum(jnp.where(onehot, px, 0.0), axis=1, keepdims=True)
        sely = jnp.sum(jnp.where(onehot, py, 0.0), axis=1, keepdims=True)
        selz = jnp.sum(jnp.where(onehot, pz, 0.0), axis=1, keepdims=True)
        d = (px - selx) ** 2 + (py - sely) ** 2 + (pz - selz) ** 2
        dists = jnp.minimum(dists, d)
        here = iota_s == i
        idx_acc = jnp.where(here, nxt, idx_acc)
        p1x = jnp.where(here, selx, p1x)
        p1y = jnp.where(here, sely, p1y)
        p1z = jnp.where(here, selz, p1z)
        return dists, idx_acc, p1x, p1y, p1z

    carry = (dists, idx_acc, p1x, p1y, p1z)
    carry = lax.fori_loop(1, n_sample, body, carry)
    _, idx_acc, p1x, p1y, p1z = carry
    return idx_acc, p1x, p1y, p1z


def _fps_kernel(px_ref, py_ref, pz_ref,
                idx1_ref, p1x_ref, p1y_ref, p1z_ref,
                idx2_ref, p2x_ref, p2y_ref, p2z_ref):
    px = px_ref[...]
    py = py_ref[...]
    pz = pz_ref[...]
    idx1, p1x, p1y, p1z = _fps_body(px, py, pz, P, S1)
    idx1_ref[...] = idx1
    p1x_ref[...] = p1x
    p1y_ref[...] = p1y
    p1z_ref[...] = p1z
    idx2, p2x, p2y, p2z = _fps_body(p1x, p1y, p1z, S1, S2)
    idx2_ref[...] = idx2
    p2x_ref[...] = p2x
    p2y_ref[...] = p2y
    p2z_ref[...] = p2z


def _run_fps(px, py, pz):
    out_shape = (
        jax.ShapeDtypeStruct((B, S1), jnp.int32),
        jax.ShapeDtypeStruct((B, S1), jnp.float32),
        jax.ShapeDtypeStruct((B, S1), jnp.float32),
        jax.ShapeDtypeStruct((B, S1), jnp.float32),
        jax.ShapeDtypeStruct((B, S2), jnp.int32),
        jax.ShapeDtypeStruct((B, S2), jnp.float32),
        jax.ShapeDtypeStruct((B, S2), jnp.float32),
        jax.ShapeDtypeStruct((B, S2), jnp.float32),
    )
    return pl.pallas_call(_fps_kernel, out_shape=out_shape)(px, py, pz)


_ROWS_BLK = 4096


def _pc1_kernel(rel_ref, w1_ref, b1_ref, w2_ref, b2_ref,
                w3_ref, b3_ref, out_ref):
    h = jnp.dot(rel_ref[...], w1_ref[...], preferred_element_type=jnp.float32)
    h = jnp.maximum(h + b1_ref[...], 0.0)
    h = jnp.dot(h, w2_ref[...], preferred_element_type=jnp.float32)
    h = jnp.maximum(h + b2_ref[...], 0.0)
    h = jnp.dot(h, w3_ref[...], preferred_element_type=jnp.float32)
    h = h + b3_ref[...]
    out_ref[...] = jnp.max(h.reshape(_ROWS_BLK // K, K, h.shape[-1]), axis=1)


def _run_pc1(rel, layers):
    (w1, b1), (w2, b2), (w3, b3) = layers
    n = rel.shape[0]
    grid = n // _ROWS_BLK
    qblk = _ROWS_BLK // K
    co = w3.shape[1]
    full = lambda a: pl.BlockSpec(a.shape, lambda i: (0,) * a.ndim)
    return pl.pallas_call(
        _pc1_kernel,
        grid=(grid,),
        in_specs=[
            pl.BlockSpec((_ROWS_BLK, 3), lambda i: (i, 0)),
            full(w1), full(b1.reshape(1, -1)),
            full(w2), full(b2.reshape(1, -1)),
            full(w3), full(b3.reshape(1, -1)),
        ],
        out_specs=pl.BlockSpec((qblk, co), lambda i: (i, 0)),
        out_shape=jax.ShapeDtypeStruct((n // K, co), jnp.float32),
    )(rel, w1, b1.reshape(1, -1), w2, b2.reshape(1, -1),
      w3, b3.reshape(1, -1))


def _pc2_kernel(xj_ref, rel_ref, w1a_ref, w1b_ref, b1_ref,
                w2_ref, b2_ref, w3_ref, b3_ref, out_ref):
    h = jnp.dot(xj_ref[...], w1a_ref[...], preferred_element_type=jnp.float32)
    h = h + jnp.dot(rel_ref[...], w1b_ref[...],
                    preferred_element_type=jnp.float32)
    h = jnp.maximum(h + b1_ref[...], 0.0)
    h = jnp.dot(h, w2_ref[...], preferred_element_type=jnp.float32)
    h = jnp.maximum(h + b2_ref[...], 0.0)
    h = jnp.dot(h, w3_ref[...], preferred_element_type=jnp.float32)
    h = h + b3_ref[...]
    out_ref[...] = jnp.max(h.reshape(_ROWS_BLK // K, K, h.shape[-1]), axis=1)


def _run_pc2(xj, rel, layers):
    (w1, b1), (w2, b2), (w3, b3) = layers
    ci = xj.shape[1]
    w1a, w1b = w1[:ci], w1[ci:]
    n = xj.shape[0]
    grid = n // _ROWS_BLK
    qblk = _ROWS_BLK // K
    co = w3.shape[1]
    full = lambda a: pl.BlockSpec(a.shape, lambda i: (0,) * a.ndim)
    return pl.pallas_call(
        _pc2_kernel,
        grid=(grid,),
        in_specs=[
            pl.BlockSpec((_ROWS_BLK, ci), lambda i: (i, 0)),
            pl.BlockSpec((_ROWS_BLK, 3), lambda i: (i, 0)),
            full(w1a), full(w1b), full(b1.reshape(1, -1)),
            full(w2), full(b2.reshape(1, -1)),
            full(w3), full(b3.reshape(1, -1)),
        ],
        out_specs=pl.BlockSpec((qblk, co), lambda i: (i, 0)),
        out_shape=jax.ShapeDtypeStruct((n // K, co), jnp.float32),
    )(xj, rel, w1a, w1b, b1.reshape(1, -1), w2, b2.reshape(1, -1),
      w3, b3.reshape(1, -1))


def _glob_kernel(feat_ref, w1_ref, b1_ref, w2_ref, b2_ref, w3_ref, b3_ref,
                 out_ref):
    h = jnp.dot(feat_ref[...], w1_ref[...], preferred_element_type=jnp.float32)
    h = jnp.maximum(h + b1_ref[...], 0.0)
    h = jnp.dot(h, w2_ref[...], preferred_element_type=jnp.float32)
    h = jnp.maximum(h + b2_ref[...], 0.0)
    h = jnp.dot(h, w3_ref[...], preferred_element_type=jnp.float32)
    h = h + b3_ref[...]
    out_ref[...] = jnp.max(h, axis=0, keepdims=True)[None]


def _run_glob(feat, layers):
    (w1, b1), (w2, b2), (w3, b3) = layers
    ci = feat.shape[1]
    co = w3.shape[1]
    full = lambda a: pl.BlockSpec(a.shape, lambda i: (0,) * a.ndim)
    return pl.pallas_call(
        _glob_kernel,
        grid=(B,),
        in_specs=[
            pl.BlockSpec((S2, ci), lambda i: (i, 0)),
            full(w1), full(b1.reshape(1, -1)),
            full(w2), full(b2.reshape(1, -1)),
            full(w3), full(b3.reshape(1, -1)),
        ],
        out_specs=pl.BlockSpec((1, 1, co), lambda i: (i, 0, 0)),
        out_shape=jax.ShapeDtypeStruct((B, 1, co), jnp.float32),
    )(feat, w1, b1.reshape(1, -1), w2, b2.reshape(1, -1), w3,
      b3.reshape(1, -1)).reshape(B, co)


def _head_kernel(g_ref, w1_ref, b1_ref, w2_ref, b2_ref, out_ref):
    h = jnp.dot(g_ref[...], w1_ref[...], preferred_element_type=jnp.float32)
    h = jnp.maximum(h + b1_ref[...], 0.0)
    h = jnp.dot(h, w2_ref[...], preferred_element_type=jnp.float32)
    h = h + b2_ref[...]
    m = jnp.max(h, axis=1, keepdims=True)
    e = jnp.exp(h - m)
    out_ref[...] = (h - m) - jnp.log(jnp.sum(e, axis=1, keepdims=True))


def _run_head(g, layers):
    (w1, b1), (w2, b2) = layers
    return pl.pallas_call(
        _head_kernel,
        out_shape=jax.ShapeDtypeStruct((B, NUM_CLASS), jnp.float32),
    )(g, w1, b1.reshape(1, -1), w2, b2.reshape(1, -1))


_BUF = K + 2 * SC_L
_NBKT = 16


def _bucket_of(z16):
    b = ((z16 + 1.0) * (_NBKT / 2.0)).astype(jnp.int32)
    return jnp.clip(b, 0, _NBKT - 1)


def _build_zbuckets(pxv, pyv, pzv, ppxv, ppyv, ppzv, ppiv, startsv, n_pts,
                    iota16):
    n_chunks = n_pts // SC_L
    cnt = jnp.int32(0)
    for k in range(_NBKT):
        plsc.store_scatter(startsv, [jnp.full((SC_L,), k, jnp.int32)],
                           jnp.full((SC_L,), cnt, jnp.int32))

        def chunk(c, cnt, k=k):
            z = pzv[pl.ds(c * SC_L, SC_L)]
            mask = _bucket_of(z) == k
            mi = mask.astype(jnp.int32)
            cums = plsc.cumsum(mi)
            slots = jnp.where(mask, cnt + cums - mi, n_pts + iota16)
            plsc.store_scatter(ppiv, [slots], iota16 + c * SC_L)
            return cnt + cums[SC_L - 1]

        cnt = lax.fori_loop(0, n_chunks, chunk, cnt)
    plsc.store_scatter(startsv, [jnp.full((SC_L,), _NBKT, jnp.int32)],
                       jnp.full((SC_L,), n_pts, jnp.int32))

    def fill(c, _):
        idxv = ppiv[pl.ds(c * SC_L, SC_L)]
        ppxv[pl.ds(c * SC_L, SC_L)] = plsc.load_gather(pxv, [idxv])
        ppyv[pl.ds(c * SC_L, SC_L)] = plsc.load_gather(pyv, [idxv])
        ppzv[pl.ds(c * SC_L, SC_L)] = plsc.load_gather(pzv, [idxv])
        return 0

    lax.fori_loop(0, n_chunks, fill, 0)


def _search_row(ppxv, ppyv, ppzv, ppiv, bufv, qxs, qys, qzs, selfs, rr,
                c0, c1, iota16):
    trash = K + SC_L + iota16
    for s in range(_BUF // SC_L):
        bufv[pl.ds(s * SC_L, SC_L)] = selfs

    def chunk(c, cnt):
        base = c * SC_L
        dx = ppxv[pl.ds(base, SC_L)] - qxs
        dy = ppyv[pl.ds(base, SC_L)] - qys
        dz = ppzv[pl.ds(base, SC_L)] - qzs
        dsq = dx * dx + dy * dy + dz * dz
        mask = dsq <= rr
        mi = mask.astype(jnp.int32)
        cums = plsc.cumsum(mi)
        slots = jnp.where(mask, cnt + cums - mi, trash)
        plsc.store_scatter(bufv, [slots], ppiv[pl.ds(base, SC_L)])
        return jnp.minimum(cnt + cums[SC_L - 1], K)

    lax.fori_loop(c0, c1, chunk, 0)


_QW1 = S1 // 2


def _rs1_kernel(px_hbm, py_hbm, pz_hbm, qx_hbm, qy_hbm, qz_hbm, self_hbm,
                rx_hbm, ry_hbm, rz_hbm,
                pxv, pyv, pzv, qxv, qyv, qzv, selfv, bufv, rxv, ryv, rzv,
                ppxv, ppyv, ppzv, ppiv, startsv):
    wid = lax.axis_index("s") * SC_NC + lax.axis_index("c")
    b = wid // 2
    h = wid % 2
    pltpu.sync_copy(px_hbm.at[b], pxv)
    pltpu.sync_copy(py_hbm.at[b], pyv)
    pltpu.sync_copy(pz_hbm.at[b], pzv)
    q0 = h * _QW1
    pltpu.sync_copy(qx_hbm.at[b, pl.ds(q0, _QW1)], qxv)
    pltpu.sync_copy(qy_hbm.at[b, pl.ds(q0, _QW1)], qyv)
    pltpu.sync_copy(qz_hbm.at[b, pl.ds(q0, _QW1)], qzv)
    pltpu.sync_copy(self_hbm.at[b, pl.ds(q0, _QW1)], selfv)
    iota16 = lax.broadcasted_iota(jnp.int32, (SC_L,), 0)
    r = 0.2
    rr = jnp.float32(r * r)
    _build_zbuckets(pxv, pyv, pzv, ppxv, ppyv, ppzv, ppiv, startsv, P,
                    iota16)

    def qchunk(qb, _):
        qx16 = qxv[pl.ds(qb * SC_L, SC_L)]
        qy16 = qyv[pl.ds(qb * SC_L, SC_L)]
        qz16 = qzv[pl.ds(qb * SC_L, SC_L)]
        self16 = selfv[pl.ds(qb * SC_L, SC_L)]
        s16 = plsc.load_gather(startsv, [_bucket_of(qz16 - r)])
        e16 = plsc.load_gather(startsv, [_bucket_of(qz16 + r) + 1])
        for j in range(SC_L):
            qi = qb * SC_L + j
            qxs = jnp.full((SC_L,), qx16[j], jnp.float32)
            qys = jnp.full((SC_L,), qy16[j], jnp.float32)
            qzs = jnp.full((SC_L,), qz16[j], jnp.float32)
            selfs = jnp.full((SC_L,), self16[j], jnp.int32)
            c0 = lax.shift_right_logical(s16[j], 4)
            c1 = lax.shift_right_logical(e16[j] + (SC_L - 1), 4)
            _search_row(ppxv, ppyv, ppzv, ppiv, bufv, qxs, qys, qzs, selfs,
                        rr, c0, c1, iota16)
            for s in range(K // SC_L):
                idxv = bufv[pl.ds(s * SC_L, SC_L)]
                rxv[qi, pl.ds(s * SC_L, SC_L)] = (
                    plsc.load_gather(pxv, [idxv]) - qxs)
                ryv[qi, pl.ds(s * SC_L, SC_L)] = (
                    plsc.load_gather(pyv, [idxv]) - qys)
                rzv[qi, pl.ds(s * SC_L, SC_L)] = (
                    plsc.load_gather(pzv, [idxv]) - qzs)
        return 0

    lax.fori_loop(0, _QW1 // SC_L, qchunk, 0)
    pltpu.sync_copy(rxv, rx_hbm.at[b, pl.ds(q0, _QW1)])
    pltpu.sync_copy(ryv, ry_hbm.at[b, pl.ds(q0, _QW1)])
    pltpu.sync_copy(rzv, rz_hbm.at[b, pl.ds(q0, _QW1)])


def _run_rs1(px, py, pz, qx, qy, qz, self_idx):
    mesh = plsc.VectorSubcoreMesh(core_axis_name="c", subcore_axis_name="s",
                                  num_cores=SC_NC, num_subcores=SC_NS)
    f32 = jnp.float32
    out_type = tuple(jax.ShapeDtypeStruct((B, S1, K), f32) for _ in range(3))
    fn = pl.kernel(
        _rs1_kernel,
        out_type=out_type,
        mesh=mesh,
        scratch_types=[
            pltpu.VMEM((P,), f32), pltpu.VMEM((P,), f32),
            pltpu.VMEM((P,), f32),
            pltpu.VMEM((_QW1,), f32), pltpu.VMEM((_QW1,), f32),
            pltpu.VMEM((_QW1,), f32),
            pltpu.VMEM((_QW1,), jnp.int32),
            pltpu.VMEM((_BUF,), jnp.int32),
            pltpu.VMEM((_QW1, K), f32), pltpu.VMEM((_QW1, K), f32),
            pltpu.VMEM((_QW1, K), f32),
            pltpu.VMEM((P,), f32), pltpu.VMEM((P,), f32),
            pltpu.VMEM((P,), f32),
            pltpu.VMEM((P + SC_L,), jnp.int32),
            pltpu.VMEM((2 * SC_L,), jnp.int32),
        ],
        compiler_params=pltpu.CompilerParams(needs_layout_passes=False),
    )
    return fn(px, py, pz, qx, qy, qz, self_idx)


_QW2 = S2 // 2
_GRP = 4


def _rs2_kernel(px_hbm, py_hbm, pz_hbm, qx_hbm, qy_hbm, qz_hbm, self_hbm,
                x1_hbm,
                rx_hbm, ry_hbm, rz_hbm, xj_hbm,
                pxv, pyv, pzv, qxv, qyv, qzv, selfv, bufv, rxv, ryv, rzv,
                idxg0, idxg1, rows_v0, rows_v1, sem0, sem1,
                ppxv, ppyv, ppzv, ppiv, startsv):
    wid = lax.axis_index("s") * SC_NC + lax.axis_index("c")
    b = wid // 2
    h = wid % 2
    pltpu.sync_copy(px_hbm.at[b], pxv)
    pltpu.sync_copy(py_hbm.at[b], pyv)
    pltpu.sync_copy(pz_hbm.at[b], pzv)
    q0 = h * _QW2
    pltpu.sync_copy(qx_hbm.at[b, pl.ds(q0, _QW2)], qxv)
    pltpu.sync_copy(qy_hbm.at[b, pl.ds(q0, _QW2)], qyv)
    pltpu.sync_copy(qz_hbm.at[b, pl.ds(q0, _QW2)], qzv)
    pltpu.sync_copy(self_hbm.at[b, pl.ds(q0, _QW2)], selfv)
    iota16 = lax.broadcasted_iota(jnp.int32, (SC_L,), 0)
    r = 0.4
    rr = jnp.float32(r * r)
    row_base = jnp.int32(b * S2 + q0)
    _build_zbuckets(pxv, pyv, pzv, ppxv, ppyv, ppzv, ppiv, startsv, S1,
                    iota16)

    idxgs = (idxg0, idxg1)
    rows = (rows_v0, rows_v1)
    sems = (sem0, sem1)
    n_grp = SC_L // _GRP

    def qchunk(qb, _):
        qx16 = qxv[pl.ds(qb * SC_L, SC_L)]
        qy16 = qyv[pl.ds(qb * SC_L, SC_L)]
        qz16 = qzv[pl.ds(qb * SC_L, SC_L)]
        self16 = selfv[pl.ds(qb * SC_L, SC_L)]
        s16 = plsc.load_gather(startsv, [_bucket_of(qz16 - r)])
        e16 = plsc.load_gather(startsv, [_bucket_of(qz16 + r) + 1])
        copies = []
        for gg in range(n_grp):
            idxg = idxgs[gg % 2]
            for j in range(_GRP):
                lane = gg * _GRP + j
                qi = qb * SC_L + lane
                qxs = jnp.full((SC_L,), qx16[lane], jnp.float32)
                qys = jnp.full((SC_L,), qy16[lane], jnp.float32)
                qzs = jnp.full((SC_L,), qz16[lane], jnp.float32)
                selfs = jnp.full((SC_L,), self16[lane], jnp.int32)
                c0 = lax.shift_right_logical(s16[lane], 4)
                c1 = lax.shift_right_logical(e16[lane] + (SC_L - 1), 4)
                _search_row(ppxv, ppyv, ppzv, ppiv, bufv, qxs, qys, qzs,
                            selfs, rr, c0, c1, iota16)
                for s in range(K // SC_L):
                    idxv = bufv[pl.ds(s * SC_L, SC_L)]
                    rxv[qi, pl.ds(s * SC_L, SC_L)] = (
                        plsc.load_gather(pxv, [idxv]) - qxs)
                    ryv[qi, pl.ds(s * SC_L, SC_L)] = (
                        plsc.load_gather(pyv, [idxv]) - qys)
                    rzv[qi, pl.ds(s * SC_L, SC_L)] = (
                        plsc.load_gather(pzv, [idxv]) - qzs)
                    idxg[pl.ds(j * K + s * SC_L, SC_L)] = idxv + b * S1
            copies.append(
                pltpu.async_copy(x1_hbm.at[idxg], rows[gg % 2],
                                 sems[gg % 2]))
            if gg >= 1:
                copies[gg - 1].wait()
                row0 = row_base + qb * SC_L + (gg - 1) * _GRP
                pltpu.sync_copy(rows[(gg - 1) % 2],
                                xj_hbm.at[pl.ds(row0 * K, _GRP * K)])
        copies[n_grp - 1].wait()
        row0 = row_base + qb * SC_L + (n_grp - 1) * _GRP
        pltpu.sync_copy(rows[(n_grp - 1) % 2],
                        xj_hbm.at[pl.ds(row0 * K, _GRP * K)])
        return 0

    lax.fori_loop(0, _QW2 // SC_L, qchunk, 0)
    pltpu.sync_copy(rxv, rx_hbm.at[b, pl.ds(q0, _QW2)])
    pltpu.sync_copy(ryv, ry_hbm.at[b, pl.ds(q0, _QW2)])
    pltpu.sync_copy(rzv, rz_hbm.at[b, pl.ds(q0, _QW2)])


def _run_rs2(px, py, pz, qx, qy, qz, self_idx, x1):
    mesh = plsc.VectorSubcoreMesh(core_axis_name="c", subcore_axis_name="s",
                                  num_cores=SC_NC, num_subcores=SC_NS)
    f32 = jnp.float32
    out_type = (
        jax.ShapeDtypeStruct((B, S2, K), f32),
        jax.ShapeDtypeStruct((B, S2, K), f32),
        jax.ShapeDtypeStruct((B, S2, K), f32),
        jax.ShapeDtypeStruct((B * S2 * K, 128), f32),
    )
    fn = pl.kernel(
        _rs2_kernel,
        out_type=out_type,
        mesh=mesh,
        scratch_types=[
            pltpu.VMEM((S1,), f32), pltpu.VMEM((S1,), f32),
            pltpu.VMEM((S1,), f32),
            pltpu.VMEM((_QW2,), f32), pltpu.VMEM((_QW2,), f32),
            pltpu.VMEM((_QW2,), f32),
            pltpu.VMEM((_QW2,), jnp.int32),
            pltpu.VMEM((_BUF,), jnp.int32),
            pltpu.VMEM((_QW2, K), f32), pltpu.VMEM((_QW2, K), f32),
            pltpu.VMEM((_QW2, K), f32),
            pltpu.VMEM((_GRP * K,), jnp.int32),
            pltpu.VMEM((_GRP * K,), jnp.int32),
            pltpu.VMEM((_GRP * K, 128), f32),
            pltpu.VMEM((_GRP * K, 128), f32),
            pltpu.SemaphoreType.DMA,
            pltpu.SemaphoreType.DMA,
            pltpu.VMEM((S1,), f32), pltpu.VMEM((S1,), f32),
            pltpu.VMEM((S1,), f32),
            pltpu.VMEM((S1 + SC_L,), jnp.int32),
            pltpu.VMEM((2 * SC_L,), jnp.int32),
        ],
        compiler_params=pltpu.CompilerParams(needs_layout_passes=False),
    )
    return fn(px, py, pz, qx, qy, qz, self_idx, x1)


def kernel(pos, batch, params):
    del batch
    pos = pos.reshape(B, P, 3)
    px, py, pz = pos[:, :, 0], pos[:, :, 1], pos[:, :, 2]
    (idx1, p1x, p1y, p1z, idx2, p2x, p2y, p2z) = _run_fps(px, py, pz)

    rx1, ry1, rz1 = _run_rs1(px, py, pz, p1x, p1y, p1z, idx1)
    rel1 = jnp.stack(
        [rx1.reshape(-1), ry1.reshape(-1), rz1.reshape(-1)], axis=-1)
    x1 = _run_pc1(rel1, params['sa1'])

    rx2, ry2, rz2, xj2 = _run_rs2(p1x, p1y, p1z, p2x, p2y, p2z, idx2, x1)
    rel2 = jnp.stack(
        [rx2.reshape(-1), ry2.reshape(-1), rz2.reshape(-1)], axis=-1)
    x2 = _run_pc2(xj2, rel2, params['sa2'])

    pos2 = jnp.stack([p2x, p2y, p2z], axis=-1)
    feat = jnp.concatenate([x2, pos2.reshape(B * S2, 3)], axis=-1)
    g = _run_glob(feat, params['sa3'])
    return _run_head(g, params['head'])


# speedup vs baseline: 1.4601x; 1.0414x over previous
import functools

import jax
import jax.numpy as jnp
from jax import lax
from jax.experimental import pallas as pl
from jax.experimental.pallas import tpu as pltpu
from jax.experimental.pallas import tpu_sc as plsc

B = 16
P = 1024
S1 = 512
S2 = 128
K = 64
NUM_CLASS = 10

SC_NC = 2
SC_NS = 16
SC_NW = SC_NC * SC_NS
SC_L = 16


_FPS_B = B // 2


def _fps_body(px, py, pz, n_pts, n_sample):
    nb = px.shape[0]
    iota_p = lax.broadcasted_iota(jnp.int32, (nb, n_pts), 1)
    iota_s = lax.broadcasted_iota(jnp.int32, (nb, n_sample), 1)

    selx0 = px[:, 0:1]
    sely0 = py[:, 0:1]
    selz0 = pz[:, 0:1]
    dists = (px - selx0) ** 2 + (py - sely0) ** 2 + (pz - selz0) ** 2

    idx_acc = jnp.zeros((nb, n_sample), jnp.int32)
    p1x = jnp.where(iota_s == 0, selx0, 0.0)
    p1y = jnp.where(iota_s == 0, sely0, 0.0)
    p1z = jnp.where(iota_s == 0, selz0, 0.0)

    def body(i, carry):
        dists, idx_acc, p1x, p1y, p1z = carry
        m = jnp.max(dists, axis=1, keepdims=True)
        cand = jnp.where(dists == m, iota_p, n_pts * 2)
        nxt = jnp.min(cand, axis=1, keepdims=True)
        onehot = iota_p == nxt
        selx = jnp.sum(jnp.where(onehot, px, 0.0), axis=1, keepdims=True)
        sely = jnp.sum(jnp.where(onehot, py, 0.0), axis=1, keepdims=True)
        selz = jnp.sum(jnp.where(onehot, pz, 0.0), axis=1, keepdims=True)
        d = (px - selx) ** 2 + (py - sely) ** 2 + (pz - selz) ** 2
        dists = jnp.minimum(dists, d)
        here = iota_s == i
        idx_acc = jnp.where(here, nxt, idx_acc)
        p1x = jnp.where(here, selx, p1x)
        p1y = jnp.where(here, sely, p1y)
        p1z = jnp.where(here, selz, p1z)
        return dists, idx_acc, p1x, p1y, p1z

    carry = (dists, idx_acc, p1x, p1y, p1z)
    carry = lax.fori_loop(1, n_sample, body, carry)
    _, idx_acc, p1x, p1y, p1z = carry
    return idx_acc, p1x, p1y, p1z


def _fps_kernel(px_ref, py_ref, pz_ref,
                idx1_ref, p1x_ref, p1y_ref, p1z_ref,
                idx2_ref, p2x_ref, p2y_ref, p2z_ref):
    px = px_ref[...]
    py = py_ref[...]
    pz = pz_ref[...]
    idx1, p1x, p1y, p1z = _fps_body(px, py, pz, P, S1)
    idx1_ref[...] = idx1
    p1x_ref[...] = p1x
    p1y_ref[...] = p1y
    p1z_ref[...] = p1z
    idx2, p2x, p2y, p2z = _fps_body(p1x, p1y, p1z, S1, S2)
    idx2_ref[...] = idx2
    p2x_ref[...] = p2x
    p2y_ref[...] = p2y
    p2z_ref[...] = p2z


def _run_fps(px, py, pz):
    out_shape = (
        jax.ShapeDtypeStruct((B, S1), jnp.int32),
        jax.ShapeDtypeStruct((B, S1), jnp.float32),
        jax.ShapeDtypeStruct((B, S1), jnp.float32),
        jax.ShapeDtypeStruct((B, S1), jnp.float32),
        jax.ShapeDtypeStruct((B, S2), jnp.int32),
        jax.ShapeDtypeStruct((B, S2), jnp.float32),
        jax.ShapeDtypeStruct((B, S2), jnp.float32),
        jax.ShapeDtypeStruct((B, S2), jnp.float32),
    )
    return pl.pallas_call(_fps_kernel, out_shape=out_shape)(px, py, pz)


_ROWS_BLK = 8192


def _pc1_kernel(rel_ref, w1_ref, b1_ref, w2_ref, b2_ref,
                w3_ref, b3_ref, out_ref):
    h = jnp.dot(rel_ref[...], w1_ref[...], preferred_element_type=jnp.float32)
    h = jnp.maximum(h + b1_ref[...], 0.0)
    h = jnp.dot(h, w2_ref[...], preferred_element_type=jnp.float32)
    h = jnp.maximum(h + b2_ref[...], 0.0)
    h = jnp.dot(h, w3_ref[...], preferred_element_type=jnp.float32)
    h = h + b3_ref[...]
    out_ref[...] = jnp.max(h.reshape(_ROWS_BLK // K, K, h.shape[-1]), axis=1)


def _run_pc1(rel, layers):
    (w1, b1), (w2, b2), (w3, b3) = layers
    n = rel.shape[0]
    grid = n // _ROWS_BLK
    qblk = _ROWS_BLK // K
    co = w3.shape[1]
    full = lambda a: pl.BlockSpec(a.shape, lambda i: (0,) * a.ndim)
    return pl.pallas_call(
        _pc1_kernel,
        grid=(grid,),
        in_specs=[
            pl.BlockSpec((_ROWS_BLK, 3), lambda i: (i, 0)),
            full(w1), full(b1.reshape(1, -1)),
            full(w2), full(b2.reshape(1, -1)),
            full(w3), full(b3.reshape(1, -1)),
        ],
        out_specs=pl.BlockSpec((qblk, co), lambda i: (i, 0)),
        out_shape=jax.ShapeDtypeStruct((n // K, co), jnp.float32),
    )(rel, w1, b1.reshape(1, -1), w2, b2.reshape(1, -1),
      w3, b3.reshape(1, -1))


def _pc2_kernel(xj_ref, rel_ref, w1a_ref, w1b_ref, b1_ref,
                w2_ref, b2_ref, w3_ref, b3_ref, out_ref):
    h = jnp.dot(xj_ref[...], w1a_ref[...], preferred_element_type=jnp.float32)
    h = h + jnp.dot(rel_ref[...], w1b_ref[...],
                    preferred_element_type=jnp.float32)
    h = jnp.maximum(h + b1_ref[...], 0.0)
    h = jnp.dot(h, w2_ref[...], preferred_element_type=jnp.float32)
    h = jnp.maximum(h + b2_ref[...], 0.0)
    h = jnp.dot(h, w3_ref[...], preferred_element_type=jnp.float32)
    h = h + b3_ref[...]
    out_ref[...] = jnp.max(h.reshape(_ROWS_BLK // K, K, h.shape[-1]), axis=1)


def _run_pc2(xj, rel, layers):
    (w1, b1), (w2, b2), (w3, b3) = layers
    ci = xj.shape[1]
    w1a, w1b = w1[:ci], w1[ci:]
    n = xj.shape[0]
    grid = n // _ROWS_BLK
    qblk = _ROWS_BLK // K
    co = w3.shape[1]
    full = lambda a: pl.BlockSpec(a.shape, lambda i: (0,) * a.ndim)
    return pl.pallas_call(
        _pc2_kernel,
        grid=(grid,),
        in_specs=[
            pl.BlockSpec((_ROWS_BLK, ci), lambda i: (i, 0)),
            pl.BlockSpec((_ROWS_BLK, 3), lambda i: (i, 0)),
            full(w1a), full(w1b), full(b1.reshape(1, -1)),
            full(w2), full(b2.reshape(1, -1)),
            full(w3), full(b3.reshape(1, -1)),
        ],
        out_specs=pl.BlockSpec((qblk, co), lambda i: (i, 0)),
        out_shape=jax.ShapeDtypeStruct((n // K, co), jnp.float32),
    )(xj, rel, w1a, w1b, b1.reshape(1, -1), w2, b2.reshape(1, -1),
      w3, b3.reshape(1, -1))


def _glob_kernel(feat_ref, w1_ref, b1_ref, w2_ref, b2_ref, w3_ref, b3_ref,
                 out_ref):
    h = jnp.dot(feat_ref[...], w1_ref[...], preferred_element_type=jnp.float32)
    h = jnp.maximum(h + b1_ref[...], 0.0)
    h = jnp.dot(h, w2_ref[...], preferred_element_type=jnp.float32)
    h = jnp.maximum(h + b2_ref[...], 0.0)
    h = jnp.dot(h, w3_ref[...], preferred_element_type=jnp.float32)
    h = h + b3_ref[...]
    out_ref[...] = jnp.max(h, axis=0, keepdims=True)[None]


def _run_glob(feat, layers):
    (w1, b1), (w2, b2), (w3, b3) = layers
    ci = feat.shape[1]
    co = w3.shape[1]
    full = lambda a: pl.BlockSpec(a.shape, lambda i: (0,) * a.ndim)
    return pl.pallas_call(
        _glob_kernel,
        grid=(B,),
        in_specs=[
            pl.BlockSpec((S2, ci), lambda i: (i, 0)),
            full(w1), full(b1.reshape(1, -1)),
            full(w2), full(b2.reshape(1, -1)),
            full(w3), full(b3.reshape(1, -1)),
        ],
        out_specs=pl.BlockSpec((1, 1, co), lambda i: (i, 0, 0)),
        out_shape=jax.ShapeDtypeStruct((B, 1, co), jnp.float32),
    )(feat, w1, b1.reshape(1, -1), w2, b2.reshape(1, -1), w3,
      b3.reshape(1, -1)).reshape(B, co)


def _head_kernel(g_ref, w1_ref, b1_ref, w2_ref, b2_ref, out_ref):
    h = jnp.dot(g_ref[...], w1_ref[...], preferred_element_type=jnp.float32)
    h = jnp.maximum(h + b1_ref[...], 0.0)
    h = jnp.dot(h, w2_ref[...], preferred_element_type=jnp.float32)
    h = h + b2_ref[...]
    m = jnp.max(h, axis=1, keepdims=True)
    e = jnp.exp(h - m)
    out_ref[...] = (h - m) - jnp.log(jnp.sum(e, axis=1, keepdims=True))


def _run_head(g, layers):
    (w1, b1), (w2, b2) = layers
    return pl.pallas_call(
        _head_kernel,
        out_shape=jax.ShapeDtypeStruct((B, NUM_CLASS), jnp.float32),
    )(g, w1, b1.reshape(1, -1), w2, b2.reshape(1, -1))


_BUF = K + 2 * SC_L
_NBKT = 16


def _bucket_of(z16):
    b = ((z16 + 1.0) * (_NBKT / 2.0)).astype(jnp.int32)
    return jnp.clip(b, 0, _NBKT - 1)


def _build_zbuckets(pxv, pyv, pzv, ppxv, ppyv, ppzv, ppiv, startsv, n_pts,
                    iota16):
    n_chunks = n_pts // SC_L
    cnt = jnp.int32(0)
    for k in range(_NBKT):
        plsc.store_scatter(startsv, [jnp.full((SC_L,), k, jnp.int32)],
                           jnp.full((SC_L,), cnt, jnp.int32))

        def chunk(c, cnt, k=k):
            z = pzv[pl.ds(c * SC_L, SC_L)]
            mask = _bucket_of(z) == k
            mi = mask.astype(jnp.int32)
            cums = plsc.cumsum(mi)
            slots = jnp.where(mask, cnt + cums - mi, n_pts + iota16)
            plsc.store_scatter(ppiv, [slots], iota16 + c * SC_L)
            return cnt + cums[SC_L - 1]

        cnt = lax.fori_loop(0, n_chunks, chunk, cnt)
    plsc.store_scatter(startsv, [jnp.full((SC_L,), _NBKT, jnp.int32)],
                       jnp.full((SC_L,), n_pts, jnp.int32))

    def fill(c, _):
        idxv = ppiv[pl.ds(c * SC_L, SC_L)]
        ppxv[pl.ds(c * SC_L, SC_L)] = plsc.load_gather(pxv, [idxv])
        ppyv[pl.ds(c * SC_L, SC_L)] = plsc.load_gather(pyv, [idxv])
        ppzv[pl.ds(c * SC_L, SC_L)] = plsc.load_gather(pzv, [idxv])
        return 0

    lax.fori_loop(0, n_chunks, fill, 0)


def _search_row(ppxv, ppyv, ppzv, ppiv, bufv, qxs, qys, qzs, selfs, rr,
                c0, c1, iota16):
    trash = K + SC_L + iota16
    for s in range(_BUF // SC_L):
        bufv[pl.ds(s * SC_L, SC_L)] = selfs

    def chunk(c, cnt):
        base = c * SC_L
        dx = ppxv[pl.ds(base, SC_L)] - qxs
        dy = ppyv[pl.ds(base, SC_L)] - qys
        dz = ppzv[pl.ds(base, SC_L)] - qzs
        dsq = dx * dx + dy * dy + dz * dz
        mask = dsq <= rr
        mi = mask.astype(jnp.int32)
        cums = plsc.cumsum(mi)
        slots = jnp.where(mask, cnt + cums - mi, trash)
        plsc.store_scatter(bufv, [slots], ppiv[pl.ds(base, SC_L)])
        return jnp.minimum(cnt + cums[SC_L - 1], K)

    lax.fori_loop(c0, c1, chunk, 0)


_QW1 = S1 // 2


def _rs1_kernel(px_hbm, py_hbm, pz_hbm, qx_hbm, qy_hbm, qz_hbm, self_hbm,
                rx_hbm, ry_hbm, rz_hbm,
                pxv, pyv, pzv, qxv, qyv, qzv, selfv, bufv, rxv, ryv, rzv,
                ppxv, ppyv, ppzv, ppiv, startsv):
    wid = lax.axis_index("s") * SC_NC + lax.axis_index("c")
    b = wid // 2
    h = wid % 2
    pltpu.sync_copy(px_hbm.at[b], pxv)
    pltpu.sync_copy(py_hbm.at[b], pyv)
    pltpu.sync_copy(pz_hbm.at[b], pzv)
    q0 = h * _QW1
    pltpu.sync_copy(qx_hbm.at[b, pl.ds(q0, _QW1)], qxv)
    pltpu.sync_copy(qy_hbm.at[b, pl.ds(q0, _QW1)], qyv)
    pltpu.sync_copy(qz_hbm.at[b, pl.ds(q0, _QW1)], qzv)
    pltpu.sync_copy(self_hbm.at[b, pl.ds(q0, _QW1)], selfv)
    iota16 = lax.broadcasted_iota(jnp.int32, (SC_L,), 0)
    r = 0.2
    rr = jnp.float32(r * r)
    _build_zbuckets(pxv, pyv, pzv, ppxv, ppyv, ppzv, ppiv, startsv, P,
                    iota16)

    def qchunk(qb, _):
        qx16 = qxv[pl.ds(qb * SC_L, SC_L)]
        qy16 = qyv[pl.ds(qb * SC_L, SC_L)]
        qz16 = qzv[pl.ds(qb * SC_L, SC_L)]
        self16 = selfv[pl.ds(qb * SC_L, SC_L)]
        s16 = plsc.load_gather(startsv, [_bucket_of(qz16 - r)])
        e16 = plsc.load_gather(startsv, [_bucket_of(qz16 + r) + 1])
        for j in range(SC_L):
            qi = qb * SC_L + j
            qxs = jnp.full((SC_L,), qx16[j], jnp.float32)
            qys = jnp.full((SC_L,), qy16[j], jnp.float32)
            qzs = jnp.full((SC_L,), qz16[j], jnp.float32)
            selfs = jnp.full((SC_L,), self16[j], jnp.int32)
            c0 = lax.shift_right_logical(s16[j], 4)
            c1 = lax.shift_right_logical(e16[j] + (SC_L - 1), 4)
            _search_row(ppxv, ppyv, ppzv, ppiv, bufv, qxs, qys, qzs, selfs,
                        rr, c0, c1, iota16)
            for s in range(K // SC_L):
                idxv = bufv[pl.ds(s * SC_L, SC_L)]
                rxv[qi, pl.ds(s * SC_L, SC_L)] = (
                    plsc.load_gather(pxv, [idxv]) - qxs)
                ryv[qi, pl.ds(s * SC_L, SC_L)] = (
                    plsc.load_gather(pyv, [idxv]) - qys)
                rzv[qi, pl.ds(s * SC_L, SC_L)] = (
                    plsc.load_gather(pzv, [idxv]) - qzs)
        return 0

    lax.fori_loop(0, _QW1 // SC_L, qchunk, 0)
    pltpu.sync_copy(rxv, rx_hbm.at[b, pl.ds(q0, _QW1)])
    pltpu.sync_copy(ryv, ry_hbm.at[b, pl.ds(q0, _QW1)])
    pltpu.sync_copy(rzv, rz_hbm.at[b, pl.ds(q0, _QW1)])


def _run_rs1(px, py, pz, qx, qy, qz, self_idx):
    mesh = plsc.VectorSubcoreMesh(core_axis_name="c", subcore_axis_name="s",
                                  num_cores=SC_NC, num_subcores=SC_NS)
    f32 = jnp.float32
    out_type = tuple(jax.ShapeDtypeStruct((B, S1, K), f32) for _ in range(3))
    fn = pl.kernel(
        _rs1_kernel,
        out_type=out_type,
        mesh=mesh,
        scratch_types=[
            pltpu.VMEM((P,), f32), pltpu.VMEM((P,), f32),
            pltpu.VMEM((P,), f32),
            pltpu.VMEM((_QW1,), f32), pltpu.VMEM((_QW1,), f32),
            pltpu.VMEM((_QW1,), f32),
            pltpu.VMEM((_QW1,), jnp.int32),
            pltpu.VMEM((_BUF,), jnp.int32),
            pltpu.VMEM((_QW1, K), f32), pltpu.VMEM((_QW1, K), f32),
            pltpu.VMEM((_QW1, K), f32),
            pltpu.VMEM((P,), f32), pltpu.VMEM((P,), f32),
            pltpu.VMEM((P,), f32),
            pltpu.VMEM((P + SC_L,), jnp.int32),
            pltpu.VMEM((2 * SC_L,), jnp.int32),
        ],
        compiler_params=pltpu.CompilerParams(needs_layout_passes=False),
    )
    return fn(px, py, pz, qx, qy, qz, self_idx)


_QW2 = S2 // 2
_GRP = 4


def _rs2_kernel(px_hbm, py_hbm, pz_hbm, qx_hbm, qy_hbm, qz_hbm, self_hbm,
                x1_hbm,
                rx_hbm, ry_hbm, rz_hbm, xj_hbm,
                pxv, pyv, pzv, qxv, qyv, qzv, selfv, bufv, rxv, ryv, rzv,
                idxg0, idxg1, rows_v0, rows_v1, sem0, sem1,
                ppxv, ppyv, ppzv, ppiv, startsv):
    wid = lax.axis_index("s") * SC_NC + lax.axis_index("c")
    b = wid // 2
    h = wid % 2
    pltpu.sync_copy(px_hbm.at[b], pxv)
    pltpu.sync_copy(py_hbm.at[b], pyv)
    pltpu.sync_copy(pz_hbm.at[b], pzv)
    q0 = h * _QW2
    pltpu.sync_copy(qx_hbm.at[b, pl.ds(q0, _QW2)], qxv)
    pltpu.sync_copy(qy_hbm.at[b, pl.ds(q0, _QW2)], qyv)
    pltpu.sync_copy(qz_hbm.at[b, pl.ds(q0, _QW2)], qzv)
    pltpu.sync_copy(self_hbm.at[b, pl.ds(q0, _QW2)], selfv)
    iota16 = lax.broadcasted_iota(jnp.int32, (SC_L,), 0)
    r = 0.4
    rr = jnp.float32(r * r)
    row_base = jnp.int32(b * S2 + q0)
    _build_zbuckets(pxv, pyv, pzv, ppxv, ppyv, ppzv, ppiv, startsv, S1,
                    iota16)

    idxgs = (idxg0, idxg1)
    rows = (rows_v0, rows_v1)
    sems = (sem0, sem1)
    n_grp = SC_L // _GRP

    def qchunk(qb, _):
        qx16 = qxv[pl.ds(qb * SC_L, SC_L)]
        qy16 = qyv[pl.ds(qb * SC_L, SC_L)]
        qz16 = qzv[pl.ds(qb * SC_L, SC_L)]
        self16 = selfv[pl.ds(qb * SC_L, SC_L)]
        s16 = plsc.load_gather(startsv, [_bucket_of(qz16 - r)])
        e16 = plsc.load_gather(startsv, [_bucket_of(qz16 + r) + 1])
        copies = []
        for gg in range(n_grp):
            idxg = idxgs[gg % 2]
            for j in range(_GRP):
                lane = gg * _GRP + j
                qi = qb * SC_L + lane
                qxs = jnp.full((SC_L,), qx16[lane], jnp.float32)
                qys = jnp.full((SC_L,), qy16[lane], jnp.float32)
                qzs = jnp.full((SC_L,), qz16[lane], jnp.float32)
                selfs = jnp.full((SC_L,), self16[lane], jnp.int32)
                c0 = lax.shift_right_logical(s16[lane], 4)
                c1 = lax.shift_right_logical(e16[lane] + (SC_L - 1), 4)
                _search_row(ppxv, ppyv, ppzv, ppiv, bufv, qxs, qys, qzs,
                            selfs, rr, c0, c1, iota16)
                for s in range(K // SC_L):
                    idxv = bufv[pl.ds(s * SC_L, SC_L)]
                    rxv[qi, pl.ds(s * SC_L, SC_L)] = (
                        plsc.load_gather(pxv, [idxv]) - qxs)
                    ryv[qi, pl.ds(s * SC_L, SC_L)] = (
                        plsc.load_gather(pyv, [idxv]) - qys)
                    rzv[qi, pl.ds(s * SC_L, SC_L)] = (
                        plsc.load_gather(pzv, [idxv]) - qzs)
                    idxg[pl.ds(j * K + s * SC_L, SC_L)] = idxv + b * S1
            copies.append(
                pltpu.async_copy(x1_hbm.at[idxg], rows[gg % 2],
                                 sems[gg % 2]))
            if gg >= 1:
                copies[gg - 1].wait()
                row0 = row_base + qb * SC_L + (gg - 1) * _GRP
                pltpu.sync_copy(rows[(gg - 1) % 2],
                                xj_hbm.at[pl.ds(row0 * K, _GRP * K)])
        copies[n_grp - 1].wait()
        row0 = row_base + qb * SC_L + (n_grp - 1) * _GRP
        pltpu.sync_copy(rows[(n_grp - 1) % 2],
                        xj_hbm.at[pl.ds(row0 * K, _GRP * K)])
        return 0

    lax.fori_loop(0, _QW2 // SC_L, qchunk, 0)
    pltpu.sync_copy(rxv, rx_hbm.at[b, pl.ds(q0, _QW2)])
    pltpu.sync_copy(ryv, ry_hbm.at[b, pl.ds(q0, _QW2)])
    pltpu.sync_copy(rzv, rz_hbm.at[b, pl.ds(q0, _QW2)])


def _run_rs2(px, py, pz, qx, qy, qz, self_idx, x1):
    mesh = plsc.VectorSubcoreMesh(core_axis_name="c", subcore_axis_name="s",
                                  num_cores=SC_NC, num_subcores=SC_NS)
    f32 = jnp.float32
    out_type = (
        jax.ShapeDtypeStruct((B, S2, K), f32),
        jax.ShapeDtypeStruct((B, S2, K), f32),
        jax.ShapeDtypeStruct((B, S2, K), f32),
        jax.ShapeDtypeStruct((B * S2 * K, 128), f32),
    )
    fn = pl.kernel(
        _rs2_kernel,
        out_type=out_type,
        mesh=mesh,
        scratch_types=[
            pltpu.VMEM((S1,), f32), pltpu.VMEM((S1,), f32),
            pltpu.VMEM((S1,), f32),
            pltpu.VMEM((_QW2,), f32), pltpu.VMEM((_QW2,), f32),
            pltpu.VMEM((_QW2,), f32),
            pltpu.VMEM((_QW2,), jnp.int32),
            pltpu.VMEM((_BUF,), jnp.int32),
            pltpu.VMEM((_QW2, K), f32), pltpu.VMEM((_QW2, K), f32),
            pltpu.VMEM((_QW2, K), f32),
            pltpu.VMEM((_GRP * K,), jnp.int32),
            pltpu.VMEM((_GRP * K,), jnp.int32),
            pltpu.VMEM((_GRP * K, 128), f32),
            pltpu.VMEM((_GRP * K, 128), f32),
            pltpu.SemaphoreType.DMA,
            pltpu.SemaphoreType.DMA,
            pltpu.VMEM((S1,), f32), pltpu.VMEM((S1,), f32),
            pltpu.VMEM((S1,), f32),
            pltpu.VMEM((S1 + SC_L,), jnp.int32),
            pltpu.VMEM((2 * SC_L,), jnp.int32),
        ],
        compiler_params=pltpu.CompilerParams(needs_layout_passes=False),
    )
    return fn(px, py, pz, qx, qy, qz, self_idx, x1)


def kernel(pos, batch, params):
    del batch
    pos = pos.reshape(B, P, 3)
    px, py, pz = pos[:, :, 0], pos[:, :, 1], pos[:, :, 2]
    (idx1, p1x, p1y, p1z, idx2, p2x, p2y, p2z) = _run_fps(px, py, pz)

    rx1, ry1, rz1 = _run_rs1(px, py, pz, p1x, p1y, p1z, idx1)
    rel1 = jnp.stack(
        [rx1.reshape(-1), ry1.reshape(-1), rz1.reshape(-1)], axis=-1)
    x1 = _run_pc1(rel1, params['sa1'])

    rx2, ry2, rz2, xj2 = _run_rs2(p1x, p1y, p1z, p2x, p2y, p2z, idx2, x1)
    rel2 = jnp.stack(
        [rx2.reshape(-1), ry2.reshape(-1), rz2.reshape(-1)], axis=-1)
    x2 = _run_pc2(xj2, rel2, params['sa2'])

    pos2 = jnp.stack([p2x, p2y, p2z], axis=-1)
    feat = jnp.concatenate([x2, pos2.reshape(B * S2, 3)], axis=-1)
    g = _run_glob(feat, params['sa3'])
    return _run_head(g, params['head'])


# speedup vs baseline: 1.4848x; 1.0169x over previous
import functools

import jax
import jax.numpy as jnp
from jax import lax
from jax.experimental import pallas as pl
from jax.experimental.pallas import tpu as pltpu
from jax.experimental.pallas import tpu_sc as plsc

B = 16
P = 1024
S1 = 512
S2 = 128
K = 64
NUM_CLASS = 10

SC_NC = 2
SC_NS = 16
SC_NW = SC_NC * SC_NS
SC_L = 16


_FPS_B = B // 2


def _fps_body(px, py, pz, n_pts, n_sample):
    nb = px.shape[0]
    iota_p = lax.broadcasted_iota(jnp.int32, (nb, n_pts), 1)
    iota_s = lax.broadcasted_iota(jnp.int32, (nb, n_sample), 1)

    selx0 = px[:, 0:1]
    sely0 = py[:, 0:1]
    selz0 = pz[:, 0:1]
    dists = (px - selx0) ** 2 + (py - sely0) ** 2 + (pz - selz0) ** 2

    idx_acc = jnp.zeros((nb, n_sample), jnp.int32)
    p1x = jnp.where(iota_s == 0, selx0, 0.0)
    p1y = jnp.where(iota_s == 0, sely0, 0.0)
    p1z = jnp.where(iota_s == 0, selz0, 0.0)

    def body(i, carry):
        dists, idx_acc, p1x, p1y, p1z = carry
        m = jnp.max(dists, axis=1, keepdims=True)
        cand = jnp.where(dists == m, iota_p, n_pts * 2)
        nxt = jnp.min(cand, axis=1, keepdims=True)
        onehot = iota_p == nxt
        selx = jnp.sum(jnp.where(onehot, px, 0.0), axis=1, keepdims=True)
        sely = jnp.sum(jnp.where(onehot, py, 0.0), axis=1, keepdims=True)
        selz = jnp.sum(jnp.where(onehot, pz, 0.0), axis=1, keepdims=True)
        d = (px - selx) ** 2 + (py - sely) ** 2 + (pz - selz) ** 2
        dists = jnp.minimum(dists, d)
        here = iota_s == i
        idx_acc = jnp.where(here, nxt, idx_acc)
        p1x = jnp.where(here, selx, p1x)
        p1y = jnp.where(here, sely, p1y)
        p1z = jnp.where(here, selz, p1z)
        return dists, idx_acc, p1x, p1y, p1z

    carry = (dists, idx_acc, p1x, p1y, p1z)
    carry = lax.fori_loop(1, n_sample, body, carry)
    _, idx_acc, p1x, p1y, p1z = carry
    return idx_acc, p1x, p1y, p1z


def _fps_kernel(px_ref, py_ref, pz_ref,
                idx1_ref, p1x_ref, p1y_ref, p1z_ref,
                idx2_ref, p2x_ref, p2y_ref, p2z_ref):
    px = px_ref[...]
    py = py_ref[...]
    pz = pz_ref[...]
    idx1, p1x, p1y, p1z = _fps_body(px, py, pz, P, S1)
    idx1_ref[...] = idx1
    p1x_ref[...] = p1x
    p1y_ref[...] = p1y
    p1z_ref[...] = p1z
    idx2, p2x, p2y, p2z = _fps_body(p1x, p1y, p1z, S1, S2)
    idx2_ref[...] = idx2
    p2x_ref[...] = p2x
    p2y_ref[...] = p2y
    p2z_ref[...] = p2z


def _run_fps(px, py, pz):
    out_shape = (
        jax.ShapeDtypeStruct((B, S1), jnp.int32),
        jax.ShapeDtypeStruct((B, S1), jnp.float32),
        jax.ShapeDtypeStruct((B, S1), jnp.float32),
        jax.ShapeDtypeStruct((B, S1), jnp.float32),
        jax.ShapeDtypeStruct((B, S2), jnp.int32),
        jax.ShapeDtypeStruct((B, S2), jnp.float32),
        jax.ShapeDtypeStruct((B, S2), jnp.float32),
        jax.ShapeDtypeStruct((B, S2), jnp.float32),
    )
    return pl.pallas_call(_fps_kernel, out_shape=out_shape)(px, py, pz)


_ROWS_BLK = 16384


def _pc1_kernel(rel_ref, w1_ref, b1_ref, w2_ref, b2_ref,
                w3_ref, b3_ref, out_ref):
    h = jnp.dot(rel_ref[...], w1_ref[...], preferred_element_type=jnp.float32)
    h = jnp.maximum(h + b1_ref[...], 0.0)
    h = jnp.dot(h, w2_ref[...], preferred_element_type=jnp.float32)
    h = jnp.maximum(h + b2_ref[...], 0.0)
    h = jnp.dot(h, w3_ref[...], preferred_element_type=jnp.float32)
    h = h + b3_ref[...]
    out_ref[...] = jnp.max(h.reshape(_ROWS_BLK // K, K, h.shape[-1]), axis=1)


def _run_pc1(rel, layers):
    (w1, b1), (w2, b2), (w3, b3) = layers
    n = rel.shape[0]
    grid = n // _ROWS_BLK
    qblk = _ROWS_BLK // K
    co = w3.shape[1]
    full = lambda a: pl.BlockSpec(a.shape, lambda i: (0,) * a.ndim)
    return pl.pallas_call(
        _pc1_kernel,
        grid=(grid,),
        in_specs=[
            pl.BlockSpec((_ROWS_BLK, 3), lambda i: (i, 0)),
            full(w1), full(b1.reshape(1, -1)),
            full(w2), full(b2.reshape(1, -1)),
            full(w3), full(b3.reshape(1, -1)),
        ],
        out_specs=pl.BlockSpec((qblk, co), lambda i: (i, 0)),
        out_shape=jax.ShapeDtypeStruct((n // K, co), jnp.float32),
    )(rel, w1, b1.reshape(1, -1), w2, b2.reshape(1, -1),
      w3, b3.reshape(1, -1))


def _pc2_kernel(xj_ref, rel_ref, w1a_ref, w1b_ref, b1_ref,
                w2_ref, b2_ref, w3_ref, b3_ref, out_ref):
    h = jnp.dot(xj_ref[...], w1a_ref[...], preferred_element_type=jnp.float32)
    h = h + jnp.dot(rel_ref[...], w1b_ref[...],
                    preferred_element_type=jnp.float32)
    h = jnp.maximum(h + b1_ref[...], 0.0)
    h = jnp.dot(h, w2_ref[...], preferred_element_type=jnp.float32)
    h = jnp.maximum(h + b2_ref[...], 0.0)
    h = jnp.dot(h, w3_ref[...], preferred_element_type=jnp.float32)
    h = h + b3_ref[...]
    out_ref[...] = jnp.max(h.reshape(_ROWS_BLK // K, K, h.shape[-1]), axis=1)


def _run_pc2(xj, rel, layers):
    (w1, b1), (w2, b2), (w3, b3) = layers
    ci = xj.shape[1]
    w1a, w1b = w1[:ci], w1[ci:]
    n = xj.shape[0]
    grid = n // _ROWS_BLK
    qblk = _ROWS_BLK // K
    co = w3.shape[1]
    full = lambda a: pl.BlockSpec(a.shape, lambda i: (0,) * a.ndim)
    return pl.pallas_call(
        _pc2_kernel,
        grid=(grid,),
        in_specs=[
            pl.BlockSpec((_ROWS_BLK, ci), lambda i: (i, 0)),
            pl.BlockSpec((_ROWS_BLK, 3), lambda i: (i, 0)),
            full(w1a), full(w1b), full(b1.reshape(1, -1)),
            full(w2), full(b2.reshape(1, -1)),
            full(w3), full(b3.reshape(1, -1)),
        ],
        out_specs=pl.BlockSpec((qblk, co), lambda i: (i, 0)),
        out_shape=jax.ShapeDtypeStruct((n // K, co), jnp.float32),
    )(xj, rel, w1a, w1b, b1.reshape(1, -1), w2, b2.reshape(1, -1),
      w3, b3.reshape(1, -1))


def _glob_kernel(feat_ref, w1_ref, b1_ref, w2_ref, b2_ref, w3_ref, b3_ref,
                 out_ref):
    h = jnp.dot(feat_ref[...], w1_ref[...], preferred_element_type=jnp.float32)
    h = jnp.maximum(h + b1_ref[...], 0.0)
    h = jnp.dot(h, w2_ref[...], preferred_element_type=jnp.float32)
    h = jnp.maximum(h + b2_ref[...], 0.0)
    h = jnp.dot(h, w3_ref[...], preferred_element_type=jnp.float32)
    h = h + b3_ref[...]
    out_ref[...] = jnp.max(h, axis=0, keepdims=True)[None]


def _run_glob(feat, layers):
    (w1, b1), (w2, b2), (w3, b3) = layers
    ci = feat.shape[1]
    co = w3.shape[1]
    full = lambda a: pl.BlockSpec(a.shape, lambda i: (0,) * a.ndim)
    return pl.pallas_call(
        _glob_kernel,
        grid=(B,),
        in_specs=[
            pl.BlockSpec((S2, ci), lambda i: (i, 0)),
            full(w1), full(b1.reshape(1, -1)),
            full(w2), full(b2.reshape(1, -1)),
            full(w3), full(b3.reshape(1, -1)),
        ],
        out_specs=pl.BlockSpec((1, 1, co), lambda i: (i, 0, 0)),
        out_shape=jax.ShapeDtypeStruct((B, 1, co), jnp.float32),
    )(feat, w1, b1.reshape(1, -1), w2, b2.reshape(1, -1), w3,
      b3.reshape(1, -1)).reshape(B, co)


def _head_kernel(g_ref, w1_ref, b1_ref, w2_ref, b2_ref, out_ref):
    h = jnp.dot(g_ref[...], w1_ref[...], preferred_element_type=jnp.float32)
    h = jnp.maximum(h + b1_ref[...], 0.0)
    h = jnp.dot(h, w2_ref[...], preferred_element_type=jnp.float32)
    h = h + b2_ref[...]
    m = jnp.max(h, axis=1, keepdims=True)
    e = jnp.exp(h - m)
    out_ref[...] = (h - m) - jnp.log(jnp.sum(e, axis=1, keepdims=True))


def _run_head(g, layers):
    (w1, b1), (w2, b2) = layers
    return pl.pallas_call(
        _head_kernel,
        out_shape=jax.ShapeDtypeStruct((B, NUM_CLASS), jnp.float32),
    )(g, w1, b1.reshape(1, -1), w2, b2.reshape(1, -1))


_BUF = K + 2 * SC_L
_NBKT = 16


def _bucket_of(z16):
    b = ((z16 + 1.0) * (_NBKT / 2.0)).astype(jnp.int32)
    return jnp.clip(b, 0, _NBKT - 1)


def _build_zbuckets(pxv, pyv, pzv, ppxv, ppyv, ppzv, ppiv, startsv, n_pts,
                    iota16):
    n_chunks = n_pts // SC_L
    cnt = jnp.int32(0)
    for k in range(_NBKT):
        plsc.store_scatter(startsv, [jnp.full((SC_L,), k, jnp.int32)],
                           jnp.full((SC_L,), cnt, jnp.int32))

        def chunk(c, cnt, k=k):
            z = pzv[pl.ds(c * SC_L, SC_L)]
            mask = _bucket_of(z) == k
            mi = mask.astype(jnp.int32)
            cums = plsc.cumsum(mi)
            slots = jnp.where(mask, cnt + cums - mi, n_pts + iota16)
            plsc.store_scatter(ppiv, [slots], iota16 + c * SC_L)
            return cnt + cums[SC_L - 1]

        cnt = lax.fori_loop(0, n_chunks, chunk, cnt)
    plsc.store_scatter(startsv, [jnp.full((SC_L,), _NBKT, jnp.int32)],
                       jnp.full((SC_L,), n_pts, jnp.int32))

    def fill(c, _):
        idxv = ppiv[pl.ds(c * SC_L, SC_L)]
        ppxv[pl.ds(c * SC_L, SC_L)] = plsc.load_gather(pxv, [idxv])
        ppyv[pl.ds(c * SC_L, SC_L)] = plsc.load_gather(pyv, [idxv])
        ppzv[pl.ds(c * SC_L, SC_L)] = plsc.load_gather(pzv, [idxv])
        return 0

    lax.fori_loop(0, n_chunks, fill, 0)


def _search_row(ppxv, ppyv, ppzv, ppiv, bufv, qxs, qys, qzs, selfs, rr,
                c0, c1, iota16):
    trash = K + SC_L + iota16
    for s in range(_BUF // SC_L):
        bufv[pl.ds(s * SC_L, SC_L)] = selfs

    def chunk(c, cnt):
        base = c * SC_L
        dx = ppxv[pl.ds(base, SC_L)] - qxs
        dy = ppyv[pl.ds(base, SC_L)] - qys
        dz = ppzv[pl.ds(base, SC_L)] - qzs
        dsq = dx * dx + dy * dy + dz * dz
        mask = dsq <= rr
        mi = mask.astype(jnp.int32)
        cums = plsc.cumsum(mi)
        slots = jnp.where(mask, cnt + cums - mi, trash)
        plsc.store_scatter(bufv, [slots], ppiv[pl.ds(base, SC_L)])
        return jnp.minimum(cnt + cums[SC_L - 1], K)

    lax.fori_loop(c0, c1, chunk, 0)


_QW1 = S1 // 2


def _rs1_kernel(px_hbm, py_hbm, pz_hbm, qx_hbm, qy_hbm, qz_hbm, self_hbm,
                rx_hbm, ry_hbm, rz_hbm,
                pxv, pyv, pzv, qxv, qyv, qzv, selfv, bufv, rxv, ryv, rzv,
                ppxv, ppyv, ppzv, ppiv, startsv):
    wid = lax.axis_index("s") * SC_NC + lax.axis_index("c")
    b = wid // 2
    h = wid % 2
    pltpu.sync_copy(px_hbm.at[b], pxv)
    pltpu.sync_copy(py_hbm.at[b], pyv)
    pltpu.sync_copy(pz_hbm.at[b], pzv)
    q0 = h * _QW1
    pltpu.sync_copy(qx_hbm.at[b, pl.ds(q0, _QW1)], qxv)
    pltpu.sync_copy(qy_hbm.at[b, pl.ds(q0, _QW1)], qyv)
    pltpu.sync_copy(qz_hbm.at[b, pl.ds(q0, _QW1)], qzv)
    pltpu.sync_copy(self_hbm.at[b, pl.ds(q0, _QW1)], selfv)
    iota16 = lax.broadcasted_iota(jnp.int32, (SC_L,), 0)
    r = 0.2
    rr = jnp.float32(r * r)
    _build_zbuckets(pxv, pyv, pzv, ppxv, ppyv, ppzv, ppiv, startsv, P,
                    iota16)

    def qchunk(qb, _):
        qx16 = qxv[pl.ds(qb * SC_L, SC_L)]
        qy16 = qyv[pl.ds(qb * SC_L, SC_L)]
        qz16 = qzv[pl.ds(qb * SC_L, SC_L)]
        self16 = selfv[pl.ds(qb * SC_L, SC_L)]
        s16 = plsc.load_gather(startsv, [_bucket_of(qz16 - r)])
        e16 = plsc.load_gather(startsv, [_bucket_of(qz16 + r) + 1])
        for j in range(SC_L):
            qi = qb * SC_L + j
            qxs = jnp.full((SC_L,), qx16[j], jnp.float32)
            qys = jnp.full((SC_L,), qy16[j], jnp.float32)
            qzs = jnp.full((SC_L,), qz16[j], jnp.float32)
            selfs = jnp.full((SC_L,), self16[j], jnp.int32)
            c0 = lax.shift_right_logical(s16[j], 4)
            c1 = lax.shift_right_logical(e16[j] + (SC_L - 1), 4)
            _search_row(ppxv, ppyv, ppzv, ppiv, bufv, qxs, qys, qzs, selfs,
                        rr, c0, c1, iota16)
            for s in range(K // SC_L):
                idxv = bufv[pl.ds(s * SC_L, SC_L)]
                rxv[qi, pl.ds(s * SC_L, SC_L)] = (
                    plsc.load_gather(pxv, [idxv]) - qxs)
                ryv[qi, pl.ds(s * SC_L, SC_L)] = (
                    plsc.load_gather(pyv, [idxv]) - qys)
                rzv[qi, pl.ds(s * SC_L, SC_L)] = (
                    plsc.load_gather(pzv, [idxv]) - qzs)
        return 0

    lax.fori_loop(0, _QW1 // SC_L, qchunk, 0)
    pltpu.sync_copy(rxv, rx_hbm.at[b, pl.ds(q0, _QW1)])
    pltpu.sync_copy(ryv, ry_hbm.at[b, pl.ds(q0, _QW1)])
    pltpu.sync_copy(rzv, rz_hbm.at[b, pl.ds(q0, _QW1)])


def _run_rs1(px, py, pz, qx, qy, qz, self_idx):
    mesh = plsc.VectorSubcoreMesh(core_axis_name="c", subcore_axis_name="s",
                                  num_cores=SC_NC, num_subcores=SC_NS)
    f32 = jnp.float32
    out_type = tuple(jax.ShapeDtypeStruct((B, S1, K), f32) for _ in range(3))
    fn = pl.kernel(
        _rs1_kernel,
        out_type=out_type,
        mesh=mesh,
        scratch_types=[
            pltpu.VMEM((P,), f32), pltpu.VMEM((P,), f32),
            pltpu.VMEM((P,), f32),
            pltpu.VMEM((_QW1,), f32), pltpu.VMEM((_QW1,), f32),
            pltpu.VMEM((_QW1,), f32),
            pltpu.VMEM((_QW1,), jnp.int32),
            pltpu.VMEM((_BUF,), jnp.int32),
            pltpu.VMEM((_QW1, K), f32), pltpu.VMEM((_QW1, K), f32),
            pltpu.VMEM((_QW1, K), f32),
            pltpu.VMEM((P,), f32), pltpu.VMEM((P,), f32),
            pltpu.VMEM((P,), f32),
            pltpu.VMEM((P + SC_L,), jnp.int32),
            pltpu.VMEM((2 * SC_L,), jnp.int32),
        ],
        compiler_params=pltpu.CompilerParams(needs_layout_passes=False),
    )
    return fn(px, py, pz, qx, qy, qz, self_idx)


_QW2 = S2 // 2
_GRP = 4


def _rs2_kernel(px_hbm, py_hbm, pz_hbm, qx_hbm, qy_hbm, qz_hbm, self_hbm,
                x1_hbm,
                rx_hbm, ry_hbm, rz_hbm, xj_hbm,
                pxv, pyv, pzv, qxv, qyv, qzv, selfv, bufv, rxv, ryv, rzv,
                idxg0, idxg1, rows_v0, rows_v1, sem0, sem1,
                ppxv, ppyv, ppzv, ppiv, startsv):
    wid = lax.axis_index("s") * SC_NC + lax.axis_index("c")
    b = wid // 2
    h = wid % 2
    pltpu.sync_copy(px_hbm.at[b], pxv)
    pltpu.sync_copy(py_hbm.at[b], pyv)
    pltpu.sync_copy(pz_hbm.at[b], pzv)
    q0 = h * _QW2
    pltpu.sync_copy(qx_hbm.at[b, pl.ds(q0, _QW2)], qxv)
    pltpu.sync_copy(qy_hbm.at[b, pl.ds(q0, _QW2)], qyv)
    pltpu.sync_copy(qz_hbm.at[b, pl.ds(q0, _QW2)], qzv)
    pltpu.sync_copy(self_hbm.at[b, pl.ds(q0, _QW2)], selfv)
    iota16 = lax.broadcasted_iota(jnp.int32, (SC_L,), 0)
    r = 0.4
    rr = jnp.float32(r * r)
    row_base = jnp.int32(b * S2 + q0)
    _build_zbuckets(pxv, pyv, pzv, ppxv, ppyv, ppzv, ppiv, startsv, S1,
                    iota16)

    idxgs = (idxg0, idxg1)
    rows = (rows_v0, rows_v1)
    sems = (sem0, sem1)
    n_grp = SC_L // _GRP

    def qchunk(qb, _):
        qx16 = qxv[pl.ds(qb * SC_L, SC_L)]
        qy16 = qyv[pl.ds(qb * SC_L, SC_L)]
        qz16 = qzv[pl.ds(qb * SC_L, SC_L)]
        self16 = selfv[pl.ds(qb * SC_L, SC_L)]
        s16 = plsc.load_gather(startsv, [_bucket_of(qz16 - r)])
        e16 = plsc.load_gather(startsv, [_bucket_of(qz16 + r) + 1])
        copies = []
        for gg in range(n_grp):
            idxg = idxgs[gg % 2]
            for j in range(_GRP):
                lane = gg * _GRP + j
                qi = qb * SC_L + lane
                qxs = jnp.full((SC_L,), qx16[lane], jnp.float32)
                qys = jnp.full((SC_L,), qy16[lane], jnp.float32)
                qzs = jnp.full((SC_L,), qz16[lane], jnp.float32)
                selfs = jnp.full((SC_L,), self16[lane], jnp.int32)
                c0 = lax.shift_right_logical(s16[lane], 4)
                c1 = lax.shift_right_logical(e16[lane] + (SC_L - 1), 4)
                _search_row(ppxv, ppyv, ppzv, ppiv, bufv, qxs, qys, qzs,
                            selfs, rr, c0, c1, iota16)
                for s in range(K // SC_L):
                    idxv = bufv[pl.ds(s * SC_L, SC_L)]
                    rxv[qi, pl.ds(s * SC_L, SC_L)] = (
                        plsc.load_gather(pxv, [idxv]) - qxs)
                    ryv[qi, pl.ds(s * SC_L, SC_L)] = (
                        plsc.load_gather(pyv, [idxv]) - qys)
                    rzv[qi, pl.ds(s * SC_L, SC_L)] = (
                        plsc.load_gather(pzv, [idxv]) - qzs)
                    idxg[pl.ds(j * K + s * SC_L, SC_L)] = idxv + b * S1
            copies.append(
                pltpu.async_copy(x1_hbm.at[idxg], rows[gg % 2],
                                 sems[gg % 2]))
            if gg >= 1:
                copies[gg - 1].wait()
                row0 = row_base + qb * SC_L + (gg - 1) * _GRP
                pltpu.sync_copy(rows[(gg - 1) % 2],
                                xj_hbm.at[pl.ds(row0 * K, _GRP * K)])
        copies[n_grp - 1].wait()
        row0 = row_base + qb * SC_L + (n_grp - 1) * _GRP
        pltpu.sync_copy(rows[(n_grp - 1) % 2],
                        xj_hbm.at[pl.ds(row0 * K, _GRP * K)])
        return 0

    lax.fori_loop(0, _QW2 // SC_L, qchunk, 0)
    pltpu.sync_copy(rxv, rx_hbm.at[b, pl.ds(q0, _QW2)])
    pltpu.sync_copy(ryv, ry_hbm.at[b, pl.ds(q0, _QW2)])
    pltpu.sync_copy(rzv, rz_hbm.at[b, pl.ds(q0, _QW2)])


def _run_rs2(px, py, pz, qx, qy, qz, self_idx, x1):
    mesh = plsc.VectorSubcoreMesh(core_axis_name="c", subcore_axis_name="s",
                                  num_cores=SC_NC, num_subcores=SC_NS)
    f32 = jnp.float32
    out_type = (
        jax.ShapeDtypeStruct((B, S2, K), f32),
        jax.ShapeDtypeStruct((B, S2, K), f32),
        jax.ShapeDtypeStruct((B, S2, K), f32),
        jax.ShapeDtypeStruct((B * S2 * K, 128), f32),
    )
    fn = pl.kernel(
        _rs2_kernel,
        out_type=out_type,
        mesh=mesh,
        scratch_types=[
            pltpu.VMEM((S1,), f32), pltpu.VMEM((S1,), f32),
            pltpu.VMEM((S1,), f32),
            pltpu.VMEM((_QW2,), f32), pltpu.VMEM((_QW2,), f32),
            pltpu.VMEM((_QW2,), f32),
            pltpu.VMEM((_QW2,), jnp.int32),
            pltpu.VMEM((_BUF,), jnp.int32),
            pltpu.VMEM((_QW2, K), f32), pltpu.VMEM((_QW2, K), f32),
            pltpu.VMEM((_QW2, K), f32),
            pltpu.VMEM((_GRP * K,), jnp.int32),
            pltpu.VMEM((_GRP * K,), jnp.int32),
            pltpu.VMEM((_GRP * K, 128), f32),
            pltpu.VMEM((_GRP * K, 128), f32),
            pltpu.SemaphoreType.DMA,
            pltpu.SemaphoreType.DMA,
            pltpu.VMEM((S1,), f32), pltpu.VMEM((S1,), f32),
            pltpu.VMEM((S1,), f32),
            pltpu.VMEM((S1 + SC_L,), jnp.int32),
            pltpu.VMEM((2 * SC_L,), jnp.int32),
        ],
        compiler_params=pltpu.CompilerParams(needs_layout_passes=False),
    )
    return fn(px, py, pz, qx, qy, qz, self_idx, x1)


def kernel(pos, batch, params):
    del batch
    pos = pos.reshape(B, P, 3)
    px, py, pz = pos[:, :, 0], pos[:, :, 1], pos[:, :, 2]
    (idx1, p1x, p1y, p1z, idx2, p2x, p2y, p2z) = _run_fps(px, py, pz)

    rx1, ry1, rz1 = _run_rs1(px, py, pz, p1x, p1y, p1z, idx1)
    rel1 = jnp.stack(
        [rx1.reshape(-1), ry1.reshape(-1), rz1.reshape(-1)], axis=-1)
    x1 = _run_pc1(rel1, params['sa1'])

    rx2, ry2, rz2, xj2 = _run_rs2(p1x, p1y, p1z, p2x, p2y, p2z, idx2, x1)
    rel2 = jnp.stack(
        [rx2.reshape(-1), ry2.reshape(-1), rz2.reshape(-1)], axis=-1)
    x2 = _run_pc2(xj2, rel2, params['sa2'])

    pos2 = jnp.stack([p2x, p2y, p2z], axis=-1)
    feat = jnp.concatenate([x2, pos2.reshape(B * S2, 3)], axis=-1)
    g = _run_glob(feat, params['sa3'])
    return _run_head(g, params['head'])


# speedup vs baseline: 1.4859x; 1.0007x over previous
import functools

import jax
import jax.numpy as jnp
from jax import lax
from jax.experimental import pallas as pl
from jax.experimental.pallas import tpu as pltpu
from jax.experimental.pallas import tpu_sc as plsc

B = 16
P = 1024
S1 = 512
S2 = 128
K = 64
NUM_CLASS = 10

SC_NC = 2
SC_NS = 16
SC_NW = SC_NC * SC_NS
SC_L = 16


_FPS_B = B // 2


def _fps_body(px, py, pz, n_pts, n_sample):
    nb = px.shape[0]
    iota_p = lax.broadcasted_iota(jnp.int32, (nb, n_pts), 1)
    iota_s = lax.broadcasted_iota(jnp.int32, (nb, n_sample), 1)

    selx0 = px[:, 0:1]
    sely0 = py[:, 0:1]
    selz0 = pz[:, 0:1]
    dists = (px - selx0) ** 2 + (py - sely0) ** 2 + (pz - selz0) ** 2

    idx_acc = jnp.zeros((nb, n_sample), jnp.int32)
    p1x = jnp.where(iota_s == 0, selx0, 0.0)
    p1y = jnp.where(iota_s == 0, sely0, 0.0)
    p1z = jnp.where(iota_s == 0, selz0, 0.0)

    def body(i, carry):
        dists, idx_acc, p1x, p1y, p1z = carry
        m = jnp.max(dists, axis=1, keepdims=True)
        cand = jnp.where(dists == m, iota_p, n_pts * 2)
        nxt = jnp.min(cand, axis=1, keepdims=True)
        onehot = iota_p == nxt
        selx = jnp.sum(jnp.where(onehot, px, 0.0), axis=1, keepdims=True)
        sely = jnp.sum(jnp.where(onehot, py, 0.0), axis=1, keepdims=True)
        selz = jnp.sum(jnp.where(onehot, pz, 0.0), axis=1, keepdims=True)
        d = (px - selx) ** 2 + (py - sely) ** 2 + (pz - selz) ** 2
        dists = jnp.minimum(dists, d)
        here = iota_s == i
        idx_acc = jnp.where(here, nxt, idx_acc)
        p1x = jnp.where(here, selx, p1x)
        p1y = jnp.where(here, sely, p1y)
        p1z = jnp.where(here, selz, p1z)
        return dists, idx_acc, p1x, p1y, p1z

    carry = (dists, idx_acc, p1x, p1y, p1z)
    carry = lax.fori_loop(1, n_sample, body, carry)
    _, idx_acc, p1x, p1y, p1z = carry
    return idx_acc, p1x, p1y, p1z


def _fps_kernel(px_ref, py_ref, pz_ref,
                idx1_ref, p1x_ref, p1y_ref, p1z_ref,
                idx2_ref, p2x_ref, p2y_ref, p2z_ref):
    px = px_ref[...]
    py = py_ref[...]
    pz = pz_ref[...]
    idx1, p1x, p1y, p1z = _fps_body(px, py, pz, P, S1)
    idx1_ref[...] = idx1
    p1x_ref[...] = p1x
    p1y_ref[...] = p1y
    p1z_ref[...] = p1z
    idx2, p2x, p2y, p2z = _fps_body(p1x, p1y, p1z, S1, S2)
    idx2_ref[...] = idx2
    p2x_ref[...] = p2x
    p2y_ref[...] = p2y
    p2z_ref[...] = p2z


def _run_fps(px, py, pz):
    out_shape = (
        jax.ShapeDtypeStruct((B, S1), jnp.int32),
        jax.ShapeDtypeStruct((B, S1), jnp.float32),
        jax.ShapeDtypeStruct((B, S1), jnp.float32),
        jax.ShapeDtypeStruct((B, S1), jnp.float32),
        jax.ShapeDtypeStruct((B, S2), jnp.int32),
        jax.ShapeDtypeStruct((B, S2), jnp.float32),
        jax.ShapeDtypeStruct((B, S2), jnp.float32),
        jax.ShapeDtypeStruct((B, S2), jnp.float32),
    )
    return pl.pallas_call(_fps_kernel, out_shape=out_shape)(px, py, pz)


_ROWS_BLK = 16384


def _pc1_kernel(rel_ref, w1_ref, b1_ref, w2_ref, b2_ref,
                w3_ref, b3_ref, out_ref):
    bf = jnp.bfloat16
    h = jnp.dot(rel_ref[...].astype(bf), w1_ref[...],
                preferred_element_type=jnp.float32)
    h = jnp.maximum(h + b1_ref[...], 0.0)
    h = jnp.dot(h.astype(bf), w2_ref[...],
                preferred_element_type=jnp.float32)
    h = jnp.maximum(h + b2_ref[...], 0.0)
    h = jnp.dot(h.astype(bf), w3_ref[...],
                preferred_element_type=jnp.float32)
    h = h + b3_ref[...]
    out_ref[...] = jnp.max(h.reshape(_ROWS_BLK // K, K, h.shape[-1]), axis=1)


def _run_pc1(rel, layers):
    (w1, b1), (w2, b2), (w3, b3) = layers
    n = rel.shape[0]
    grid = n // _ROWS_BLK
    qblk = _ROWS_BLK // K
    co = w3.shape[1]
    full = lambda a: pl.BlockSpec(a.shape, lambda i: (0,) * a.ndim)
    return pl.pallas_call(
        _pc1_kernel,
        grid=(grid,),
        in_specs=[
            pl.BlockSpec((_ROWS_BLK, 3), lambda i: (i, 0)),
            full(w1), full(b1.reshape(1, -1)),
            full(w2), full(b2.reshape(1, -1)),
            full(w3), full(b3.reshape(1, -1)),
        ],
        out_specs=pl.BlockSpec((qblk, co), lambda i: (i, 0)),
        out_shape=jax.ShapeDtypeStruct((n // K, co), jnp.float32),
    )(rel, w1.astype(jnp.bfloat16), b1.reshape(1, -1),
      w2.astype(jnp.bfloat16), b2.reshape(1, -1),
      w3.astype(jnp.bfloat16), b3.reshape(1, -1))


def _pc2_kernel(xj_ref, rel_ref, w1a_ref, w1b_ref, b1_ref,
                w2_ref, b2_ref, w3_ref, b3_ref, out_ref):
    bf = jnp.bfloat16
    h = jnp.dot(xj_ref[...].astype(bf), w1a_ref[...],
                preferred_element_type=jnp.float32)
    h = h + jnp.dot(rel_ref[...].astype(bf), w1b_ref[...],
                    preferred_element_type=jnp.float32)
    h = jnp.maximum(h + b1_ref[...], 0.0)
    h = jnp.dot(h.astype(bf), w2_ref[...],
                preferred_element_type=jnp.float32)
    h = jnp.maximum(h + b2_ref[...], 0.0)
    h = jnp.dot(h.astype(bf), w3_ref[...],
                preferred_element_type=jnp.float32)
    h = h + b3_ref[...]
    out_ref[...] = jnp.max(h.reshape(_ROWS_BLK // K, K, h.shape[-1]), axis=1)


def _run_pc2(xj, rel, layers):
    (w1, b1), (w2, b2), (w3, b3) = layers
    ci = xj.shape[1]
    w1a, w1b = w1[:ci], w1[ci:]
    n = xj.shape[0]
    grid = n // _ROWS_BLK
    qblk = _ROWS_BLK // K
    co = w3.shape[1]
    full = lambda a: pl.BlockSpec(a.shape, lambda i: (0,) * a.ndim)
    return pl.pallas_call(
        _pc2_kernel,
        grid=(grid,),
        in_specs=[
            pl.BlockSpec((_ROWS_BLK, ci), lambda i: (i, 0)),
            pl.BlockSpec((_ROWS_BLK, 3), lambda i: (i, 0)),
            full(w1a), full(w1b), full(b1.reshape(1, -1)),
            full(w2), full(b2.reshape(1, -1)),
            full(w3), full(b3.reshape(1, -1)),
        ],
        out_specs=pl.BlockSpec((qblk, co), lambda i: (i, 0)),
        out_shape=jax.ShapeDtypeStruct((n // K, co), jnp.float32),
    )(xj, rel, w1a.astype(jnp.bfloat16), w1b.astype(jnp.bfloat16),
      b1.reshape(1, -1), w2.astype(jnp.bfloat16), b2.reshape(1, -1),
      w3.astype(jnp.bfloat16), b3.reshape(1, -1))


def _glob_kernel(feat_ref, w1_ref, b1_ref, w2_ref, b2_ref, w3_ref, b3_ref,
                 out_ref):
    bf = jnp.bfloat16
    h = jnp.dot(feat_ref[...].astype(bf), w1_ref[...],
                preferred_element_type=jnp.float32)
    h = jnp.maximum(h + b1_ref[...], 0.0)
    h = jnp.dot(h.astype(bf), w2_ref[...],
                preferred_element_type=jnp.float32)
    h = jnp.maximum(h + b2_ref[...], 0.0)
    h = jnp.dot(h.astype(bf), w3_ref[...],
                preferred_element_type=jnp.float32)
    h = h + b3_ref[...]
    out_ref[...] = jnp.max(h, axis=0, keepdims=True)[None]


def _run_glob(feat, layers):
    (w1, b1), (w2, b2), (w3, b3) = layers
    ci = feat.shape[1]
    co = w3.shape[1]
    full = lambda a: pl.BlockSpec(a.shape, lambda i: (0,) * a.ndim)
    return pl.pallas_call(
        _glob_kernel,
        grid=(B,),
        in_specs=[
            pl.BlockSpec((S2, ci), lambda i: (i, 0)),
            full(w1), full(b1.reshape(1, -1)),
            full(w2), full(b2.reshape(1, -1)),
            full(w3), full(b3.reshape(1, -1)),
        ],
        out_specs=pl.BlockSpec((1, 1, co), lambda i: (i, 0, 0)),
        out_shape=jax.ShapeDtypeStruct((B, 1, co), jnp.float32),
    )(feat, w1.astype(jnp.bfloat16), b1.reshape(1, -1),
      w2.astype(jnp.bfloat16), b2.reshape(1, -1),
      w3.astype(jnp.bfloat16), b3.reshape(1, -1)).reshape(B, co)


def _head_kernel(g_ref, w1_ref, b1_ref, w2_ref, b2_ref, out_ref):
    h = jnp.dot(g_ref[...], w1_ref[...], preferred_element_type=jnp.float32)
    h = jnp.maximum(h + b1_ref[...], 0.0)
    h = jnp.dot(h, w2_ref[...], preferred_element_type=jnp.float32)
    h = h + b2_ref[...]
    m = jnp.max(h, axis=1, keepdims=True)
    e = jnp.exp(h - m)
    out_ref[...] = (h - m) - jnp.log(jnp.sum(e, axis=1, keepdims=True))


def _run_head(g, layers):
    (w1, b1), (w2, b2) = layers
    return pl.pallas_call(
        _head_kernel,
        out_shape=jax.ShapeDtypeStruct((B, NUM_CLASS), jnp.float32),
    )(g, w1, b1.reshape(1, -1), w2, b2.reshape(1, -1))


_BUF = K + 2 * SC_L
_NBKT = 16


def _bucket_of(z16):
    b = ((z16 + 1.0) * (_NBKT / 2.0)).astype(jnp.int32)
    return jnp.clip(b, 0, _NBKT - 1)


def _build_zbuckets(pxv, pyv, pzv, ppxv, ppyv, ppzv, ppiv, startsv, n_pts,
                    iota16):
    n_chunks = n_pts // SC_L
    cnt = jnp.int32(0)
    for k in range(_NBKT):
        plsc.store_scatter(startsv, [jnp.full((SC_L,), k, jnp.int32)],
                           jnp.full((SC_L,), cnt, jnp.int32))

        def chunk(c, cnt, k=k):
            z = pzv[pl.ds(c * SC_L, SC_L)]
            mask = _bucket_of(z) == k
            mi = mask.astype(jnp.int32)
            cums = plsc.cumsum(mi)
            slots = jnp.where(mask, cnt + cums - mi, n_pts + iota16)
            plsc.store_scatter(ppiv, [slots], iota16 + c * SC_L)
            return cnt + cums[SC_L - 1]

        cnt = lax.fori_loop(0, n_chunks, chunk, cnt)
    plsc.store_scatter(startsv, [jnp.full((SC_L,), _NBKT, jnp.int32)],
                       jnp.full((SC_L,), n_pts, jnp.int32))

    def fill(c, _):
        idxv = ppiv[pl.ds(c * SC_L, SC_L)]
        ppxv[pl.ds(c * SC_L, SC_L)] = plsc.load_gather(pxv, [idxv])
        ppyv[pl.ds(c * SC_L, SC_L)] = plsc.load_gather(pyv, [idxv])
        ppzv[pl.ds(c * SC_L, SC_L)] = plsc.load_gather(pzv, [idxv])
        return 0

    lax.fori_loop(0, n_chunks, fill, 0)


def _search_row(ppxv, ppyv, ppzv, ppiv, bufv, qxs, qys, qzs, selfs, rr,
                c0, c1, iota16):
    trash = K + SC_L + iota16
    for s in range(_BUF // SC_L):
        bufv[pl.ds(s * SC_L, SC_L)] = selfs

    def chunk(c, cnt):
        base = c * SC_L
        dx = ppxv[pl.ds(base, SC_L)] - qxs
        dy = ppyv[pl.ds(base, SC_L)] - qys
        dz = ppzv[pl.ds(base, SC_L)] - qzs
        dsq = dx * dx + dy * dy + dz * dz
        mask = dsq <= rr
        mi = mask.astype(jnp.int32)
        cums = plsc.cumsum(mi)
        slots = jnp.where(mask, cnt + cums - mi, trash)
        plsc.store_scatter(bufv, [slots], ppiv[pl.ds(base, SC_L)])
        return jnp.minimum(cnt + cums[SC_L - 1], K)

    lax.fori_loop(c0, c1, chunk, 0)


_QW1 = S1 // 2


def _rs1_kernel(px_hbm, py_hbm, pz_hbm, qx_hbm, qy_hbm, qz_hbm, self_hbm,
                rx_hbm, ry_hbm, rz_hbm,
                pxv, pyv, pzv, qxv, qyv, qzv, selfv, bufv, rxv, ryv, rzv,
                ppxv, ppyv, ppzv, ppiv, startsv):
    wid = lax.axis_index("s") * SC_NC + lax.axis_index("c")
    b = wid // 2
    h = wid % 2
    pltpu.sync_copy(px_hbm.at[b], pxv)
    pltpu.sync_copy(py_hbm.at[b], pyv)
    pltpu.sync_copy(pz_hbm.at[b], pzv)
    q0 = h * _QW1
    pltpu.sync_copy(qx_hbm.at[b, pl.ds(q0, _QW1)], qxv)
    pltpu.sync_copy(qy_hbm.at[b, pl.ds(q0, _QW1)], qyv)
    pltpu.sync_copy(qz_hbm.at[b, pl.ds(q0, _QW1)], qzv)
    pltpu.sync_copy(self_hbm.at[b, pl.ds(q0, _QW1)], selfv)
    iota16 = lax.broadcasted_iota(jnp.int32, (SC_L,), 0)
    r = 0.2
    rr = jnp.float32(r * r)
    _build_zbuckets(pxv, pyv, pzv, ppxv, ppyv, ppzv, ppiv, startsv, P,
                    iota16)

    def qchunk(qb, _):
        qx16 = qxv[pl.ds(qb * SC_L, SC_L)]
        qy16 = qyv[pl.ds(qb * SC_L, SC_L)]
        qz16 = qzv[pl.ds(qb * SC_L, SC_L)]
        self16 = selfv[pl.ds(qb * SC_L, SC_L)]
        s16 = plsc.load_gather(startsv, [_bucket_of(qz16 - r)])
        e16 = plsc.load_gather(startsv, [_bucket_of(qz16 + r) + 1])
        for j in range(SC_L):
            qi = qb * SC_L + j
            qxs = jnp.full((SC_L,), qx16[j], jnp.float32)
            qys = jnp.full((SC_L,), qy16[j], jnp.float32)
            qzs = jnp.full((SC_L,), qz16[j], jnp.float32)
            selfs = jnp.full((SC_L,), self16[j], jnp.int32)
            c0 = lax.shift_right_logical(s16[j], 4)
            c1 = lax.shift_right_logical(e16[j] + (SC_L - 1), 4)
            _search_row(ppxv, ppyv, ppzv, ppiv, bufv, qxs, qys, qzs, selfs,
                        rr, c0, c1, iota16)
            for s in range(K // SC_L):
                idxv = bufv[pl.ds(s * SC_L, SC_L)]
                rxv[qi, pl.ds(s * SC_L, SC_L)] = (
                    plsc.load_gather(pxv, [idxv]) - qxs)
                ryv[qi, pl.ds(s * SC_L, SC_L)] = (
                    plsc.load_gather(pyv, [idxv]) - qys)
                rzv[qi, pl.ds(s * SC_L, SC_L)] = (
                    plsc.load_gather(pzv, [idxv]) - qzs)
        return 0

    lax.fori_loop(0, _QW1 // SC_L, qchunk, 0)
    pltpu.sync_copy(rxv, rx_hbm.at[b, pl.ds(q0, _QW1)])
    pltpu.sync_copy(ryv, ry_hbm.at[b, pl.ds(q0, _QW1)])
    pltpu.sync_copy(rzv, rz_hbm.at[b, pl.ds(q0, _QW1)])


def _run_rs1(px, py, pz, qx, qy, qz, self_idx):
    mesh = plsc.VectorSubcoreMesh(core_axis_name="c", subcore_axis_name="s",
                                  num_cores=SC_NC, num_subcores=SC_NS)
    f32 = jnp.float32
    out_type = tuple(jax.ShapeDtypeStruct((B, S1, K), f32) for _ in range(3))
    fn = pl.kernel(
        _rs1_kernel,
        out_type=out_type,
        mesh=mesh,
        scratch_types=[
            pltpu.VMEM((P,), f32), pltpu.VMEM((P,), f32),
            pltpu.VMEM((P,), f32),
            pltpu.VMEM((_QW1,), f32), pltpu.VMEM((_QW1,), f32),
            pltpu.VMEM((_QW1,), f32),
            pltpu.VMEM((_QW1,), jnp.int32),
            pltpu.VMEM((_BUF,), jnp.int32),
            pltpu.VMEM((_QW1, K), f32), pltpu.VMEM((_QW1, K), f32),
            pltpu.VMEM((_QW1, K), f32),
            pltpu.VMEM((P,), f32), pltpu.VMEM((P,), f32),
            pltpu.VMEM((P,), f32),
            pltpu.VMEM((P + SC_L,), jnp.int32),
            pltpu.VMEM((2 * SC_L,), jnp.int32),
        ],
        compiler_params=pltpu.CompilerParams(needs_layout_passes=False),
    )
    return fn(px, py, pz, qx, qy, qz, self_idx)


_QW2 = S2 // 2
_GRP = 4


def _rs2_kernel(px_hbm, py_hbm, pz_hbm, qx_hbm, qy_hbm, qz_hbm, self_hbm,
                x1_hbm,
                rx_hbm, ry_hbm, rz_hbm, xj_hbm,
                pxv, pyv, pzv, qxv, qyv, qzv, selfv, bufv, rxv, ryv, rzv,
                idxg0, idxg1, rows_v0, rows_v1, sem0, sem1,
                ppxv, ppyv, ppzv, ppiv, startsv):
    wid = lax.axis_index("s") * SC_NC + lax.axis_index("c")
    b = wid // 2
    h = wid % 2
    pltpu.sync_copy(px_hbm.at[b], pxv)
    pltpu.sync_copy(py_hbm.at[b], pyv)
    pltpu.sync_copy(pz_hbm.at[b], pzv)
    q0 = h * _QW2
    pltpu.sync_copy(qx_hbm.at[b, pl.ds(q0, _QW2)], qxv)
    pltpu.sync_copy(qy_hbm.at[b, pl.ds(q0, _QW2)], qyv)
    pltpu.sync_copy(qz_hbm.at[b, pl.ds(q0, _QW2)], qzv)
    pltpu.sync_copy(self_hbm.at[b, pl.ds(q0, _QW2)], selfv)
    iota16 = lax.broadcasted_iota(jnp.int32, (SC_L,), 0)
    r = 0.4
    rr = jnp.float32(r * r)
    row_base = jnp.int32(b * S2 + q0)
    _build_zbuckets(pxv, pyv, pzv, ppxv, ppyv, ppzv, ppiv, startsv, S1,
                    iota16)

    idxgs = (idxg0, idxg1)
    rows = (rows_v0, rows_v1)
    sems = (sem0, sem1)
    n_grp = SC_L // _GRP

    def qchunk(qb, _):
        qx16 = qxv[pl.ds(qb * SC_L, SC_L)]
        qy16 = qyv[pl.ds(qb * SC_L, SC_L)]
        qz16 = qzv[pl.ds(qb * SC_L, SC_L)]
        self16 = selfv[pl.ds(qb * SC_L, SC_L)]
        s16 = plsc.load_gather(startsv, [_bucket_of(qz16 - r)])
        e16 = plsc.load_gather(startsv, [_bucket_of(qz16 + r) + 1])
        copies = []
        for gg in range(n_grp):
            idxg = idxgs[gg % 2]
            for j in range(_GRP):
                lane = gg * _GRP + j
                qi = qb * SC_L + lane
                qxs = jnp.full((SC_L,), qx16[lane], jnp.float32)
                qys = jnp.full((SC_L,), qy16[lane], jnp.float32)
                qzs = jnp.full((SC_L,), qz16[lane], jnp.float32)
                selfs = jnp.full((SC_L,), self16[lane], jnp.int32)
                c0 = lax.shift_right_logical(s16[lane], 4)
                c1 = lax.shift_right_logical(e16[lane] + (SC_L - 1), 4)
                _search_row(ppxv, ppyv, ppzv, ppiv, bufv, qxs, qys, qzs,
                            selfs, rr, c0, c1, iota16)
                for s in range(K // SC_L):
                    idxv = bufv[pl.ds(s * SC_L, SC_L)]
                    rxv[qi, pl.ds(s * SC_L, SC_L)] = (
                        plsc.load_gather(pxv, [idxv]) - qxs)
                    ryv[qi, pl.ds(s * SC_L, SC_L)] = (
                        plsc.load_gather(pyv, [idxv]) - qys)
                    rzv[qi, pl.ds(s * SC_L, SC_L)] = (
                        plsc.load_gather(pzv, [idxv]) - qzs)
                    idxg[pl.ds(j * K + s * SC_L, SC_L)] = idxv + b * S1
            copies.append(
                pltpu.async_copy(x1_hbm.at[idxg], rows[gg % 2],
                                 sems[gg % 2]))
            if gg >= 1:
                copies[gg - 1].wait()
                row0 = row_base + qb * SC_L + (gg - 1) * _GRP
                pltpu.sync_copy(rows[(gg - 1) % 2],
                                xj_hbm.at[pl.ds(row0 * K, _GRP * K)])
        copies[n_grp - 1].wait()
        row0 = row_base + qb * SC_L + (n_grp - 1) * _GRP
        pltpu.sync_copy(rows[(n_grp - 1) % 2],
                        xj_hbm.at[pl.ds(row0 * K, _GRP * K)])
        return 0

    lax.fori_loop(0, _QW2 // SC_L, qchunk, 0)
    pltpu.sync_copy(rxv, rx_hbm.at[b, pl.ds(q0, _QW2)])
    pltpu.sync_copy(ryv, ry_hbm.at[b, pl.ds(q0, _QW2)])
    pltpu.sync_copy(rzv, rz_hbm.at[b, pl.ds(q0, _QW2)])


def _run_rs2(px, py, pz, qx, qy, qz, self_idx, x1):
    mesh = plsc.VectorSubcoreMesh(core_axis_name="c", subcore_axis_name="s",
                                  num_cores=SC_NC, num_subcores=SC_NS)
    f32 = jnp.float32
    out_type = (
        jax.ShapeDtypeStruct((B, S2, K), f32),
        jax.ShapeDtypeStruct((B, S2, K), f32),
        jax.ShapeDtypeStruct((B, S2, K), f32),
        jax.ShapeDtypeStruct((B * S2 * K, 128), f32),
    )
    fn = pl.kernel(
        _rs2_kernel,
        out_type=out_type,
        mesh=mesh,
        scratch_types=[
            pltpu.VMEM((S1,), f32), pltpu.VMEM((S1,), f32),
            pltpu.VMEM((S1,), f32),
            pltpu.VMEM((_QW2,), f32), pltpu.VMEM((_QW2,), f32),
            pltpu.VMEM((_QW2,), f32),
            pltpu.VMEM((_QW2,), jnp.int32),
            pltpu.VMEM((_BUF,), jnp.int32),
            pltpu.VMEM((_QW2, K), f32), pltpu.VMEM((_QW2, K), f32),
            pltpu.VMEM((_QW2, K), f32),
            pltpu.VMEM((_GRP * K,), jnp.int32),
            pltpu.VMEM((_GRP * K,), jnp.int32),
            pltpu.VMEM((_GRP * K, 128), f32),
            pltpu.VMEM((_GRP * K, 128), f32),
            pltpu.SemaphoreType.DMA,
            pltpu.SemaphoreType.DMA,
            pltpu.VMEM((S1,), f32), pltpu.VMEM((S1,), f32),
            pltpu.VMEM((S1,), f32),
            pltpu.VMEM((S1 + SC_L,), jnp.int32),
            pltpu.VMEM((2 * SC_L,), jnp.int32),
        ],
        compiler_params=pltpu.CompilerParams(needs_layout_passes=False),
    )
    return fn(px, py, pz, qx, qy, qz, self_idx, x1)


def kernel(pos, batch, params):
    del batch
    pos = pos.reshape(B, P, 3)
    px, py, pz = pos[:, :, 0], pos[:, :, 1], pos[:, :, 2]
    (idx1, p1x, p1y, p1z, idx2, p2x, p2y, p2z) = _run_fps(px, py, pz)

    rx1, ry1, rz1 = _run_rs1(px, py, pz, p1x, p1y, p1z, idx1)
    rel1 = jnp.stack(
        [rx1.reshape(-1), ry1.reshape(-1), rz1.reshape(-1)], axis=-1)
    x1 = _run_pc1(rel1, params['sa1'])

    rx2, ry2, rz2, xj2 = _run_rs2(p1x, p1y, p1z, p2x, p2y, p2z, idx2, x1)
    rel2 = jnp.stack(
        [rx2.reshape(-1), ry2.reshape(-1), rz2.reshape(-1)], axis=-1)
    x2 = _run_pc2(xj2, rel2, params['sa2'])

    pos2 = jnp.stack([p2x, p2y, p2z], axis=-1)
    feat = jnp.concatenate([x2, pos2.reshape(B * S2, 3)], axis=-1)
    g = _run_glob(feat, params['sa3'])
    return _run_head(g, params['head'])


# speedup vs baseline: 1.4883x; 1.0016x over previous
import functools

import jax
import jax.numpy as jnp
from jax import lax
from jax.experimental import pallas as pl
from jax.experimental.pallas import tpu as pltpu
from jax.experimental.pallas import tpu_sc as plsc

B = 16
P = 1024
S1 = 512
S2 = 128
K = 64
NUM_CLASS = 10

SC_NC = 2
SC_NS = 16
SC_NW = SC_NC * SC_NS
SC_L = 16


_FPS_B = B // 2


def _fps_body(px, py, pz, n_pts, n_sample):
    nb = px.shape[0]
    iota_p = lax.broadcasted_iota(jnp.int32, (nb, n_pts), 1)
    iota_s = lax.broadcasted_iota(jnp.int32, (nb, n_sample), 1)

    selx0 = px[:, 0:1]
    sely0 = py[:, 0:1]
    selz0 = pz[:, 0:1]
    dists = (px - selx0) ** 2 + (py - sely0) ** 2 + (pz - selz0) ** 2

    idx_acc = jnp.zeros((nb, n_sample), jnp.int32)
    p1x = jnp.where(iota_s == 0, selx0, 0.0)
    p1y = jnp.where(iota_s == 0, sely0, 0.0)
    p1z = jnp.where(iota_s == 0, selz0, 0.0)

    def body(i, carry):
        dists, idx_acc, p1x, p1y, p1z = carry
        m = jnp.max(dists, axis=1, keepdims=True)
        cand = jnp.where(dists == m, iota_p, n_pts * 2)
        nxt = jnp.min(cand, axis=1, keepdims=True)
        onehot = iota_p == nxt
        selx = jnp.sum(jnp.where(onehot, px, 0.0), axis=1, keepdims=True)
        sely = jnp.sum(jnp.where(onehot, py, 0.0), axis=1, keepdims=True)
        selz = jnp.sum(jnp.where(onehot, pz, 0.0), axis=1, keepdims=True)
        d = (px - selx) ** 2 + (py - sely) ** 2 + (pz - selz) ** 2
        dists = jnp.minimum(dists, d)
        here = iota_s == i
        idx_acc = jnp.where(here, nxt, idx_acc)
        p1x = jnp.where(here, selx, p1x)
        p1y = jnp.where(here, sely, p1y)
        p1z = jnp.where(here, selz, p1z)
        return dists, idx_acc, p1x, p1y, p1z

    carry = (dists, idx_acc, p1x, p1y, p1z)
    carry = lax.fori_loop(1, n_sample, body, carry)
    _, idx_acc, p1x, p1y, p1z = carry
    return idx_acc, p1x, p1y, p1z


def _fps_kernel(px_ref, py_ref, pz_ref,
                idx1_ref, p1x_ref, p1y_ref, p1z_ref,
                idx2_ref, p2x_ref, p2y_ref, p2z_ref):
    px = px_ref[...]
    py = py_ref[...]
    pz = pz_ref[...]
    idx1, p1x, p1y, p1z = _fps_body(px, py, pz, P, S1)
    idx1_ref[...] = idx1
    p1x_ref[...] = p1x
    p1y_ref[...] = p1y
    p1z_ref[...] = p1z
    idx2, p2x, p2y, p2z = _fps_body(p1x, p1y, p1z, S1, S2)
    idx2_ref[...] = idx2
    p2x_ref[...] = p2x
    p2y_ref[...] = p2y
    p2z_ref[...] = p2z


def _run_fps(px, py, pz):
    out_shape = (
        jax.ShapeDtypeStruct((B, S1), jnp.int32),
        jax.ShapeDtypeStruct((B, S1), jnp.float32),
        jax.ShapeDtypeStruct((B, S1), jnp.float32),
        jax.ShapeDtypeStruct((B, S1), jnp.float32),
        jax.ShapeDtypeStruct((B, S2), jnp.int32),
        jax.ShapeDtypeStruct((B, S2), jnp.float32),
        jax.ShapeDtypeStruct((B, S2), jnp.float32),
        jax.ShapeDtypeStruct((B, S2), jnp.float32),
    )
    return pl.pallas_call(_fps_kernel, out_shape=out_shape)(px, py, pz)


_ROWS_BLK = 16384


def _pc1_kernel(rel_ref, w1_ref, b1_ref, w2_ref, b2_ref,
                w3_ref, b3_ref, out_ref):
    h = jnp.dot(rel_ref[...], w1_ref[...], preferred_element_type=jnp.float32)
    h = jnp.maximum(h + b1_ref[...], 0.0)
    h = jnp.dot(h, w2_ref[...], preferred_element_type=jnp.float32)
    h = jnp.maximum(h + b2_ref[...], 0.0)
    h = jnp.dot(h, w3_ref[...], preferred_element_type=jnp.float32)
    h = h + b3_ref[...]
    out_ref[...] = jnp.max(h.reshape(_ROWS_BLK // K, K, h.shape[-1]), axis=1)


def _run_pc1(rel, layers):
    (w1, b1), (w2, b2), (w3, b3) = layers
    n = rel.shape[0]
    grid = n // _ROWS_BLK
    qblk = _ROWS_BLK // K
    co = w3.shape[1]
    full = lambda a: pl.BlockSpec(a.shape, lambda i: (0,) * a.ndim)
    return pl.pallas_call(
        _pc1_kernel,
        grid=(grid,),
        in_specs=[
            pl.BlockSpec((_ROWS_BLK, 3), lambda i: (i, 0)),
            full(w1), full(b1.reshape(1, -1)),
            full(w2), full(b2.reshape(1, -1)),
            full(w3), full(b3.reshape(1, -1)),
        ],
        out_specs=pl.BlockSpec((qblk, co), lambda i: (i, 0)),
        out_shape=jax.ShapeDtypeStruct((n // K, co), jnp.float32),
    )(rel, w1, b1.reshape(1, -1), w2, b2.reshape(1, -1),
      w3, b3.reshape(1, -1))


def _pc2_kernel(xj_ref, rel_ref, w1a_ref, w1b_ref, b1_ref,
                w2_ref, b2_ref, w3_ref, b3_ref, out_ref):
    h = jnp.dot(xj_ref[...], w1a_ref[...], preferred_element_type=jnp.float32)
    h = h + jnp.dot(rel_ref[...], w1b_ref[...],
                    preferred_element_type=jnp.float32)
    h = jnp.maximum(h + b1_ref[...], 0.0)
    h = jnp.dot(h, w2_ref[...], preferred_element_type=jnp.float32)
    h = jnp.maximum(h + b2_ref[...], 0.0)
    h = jnp.dot(h, w3_ref[...], preferred_element_type=jnp.float32)
    h = h + b3_ref[...]
    out_ref[...] = jnp.max(h.reshape(_ROWS_BLK // K, K, h.shape[-1]), axis=1)


def _run_pc2(xj, rel, layers):
    (w1, b1), (w2, b2), (w3, b3) = layers
    ci = xj.shape[1]
    w1a, w1b = w1[:ci], w1[ci:]
    n = xj.shape[0]
    grid = n // _ROWS_BLK
    qblk = _ROWS_BLK // K
    co = w3.shape[1]
    full = lambda a: pl.BlockSpec(a.shape, lambda i: (0,) * a.ndim)
    return pl.pallas_call(
        _pc2_kernel,
        grid=(grid,),
        in_specs=[
            pl.BlockSpec((_ROWS_BLK, ci), lambda i: (i, 0)),
            pl.BlockSpec((_ROWS_BLK, 3), lambda i: (i, 0)),
            full(w1a), full(w1b), full(b1.reshape(1, -1)),
            full(w2), full(b2.reshape(1, -1)),
            full(w3), full(b3.reshape(1, -1)),
        ],
        out_specs=pl.BlockSpec((qblk, co), lambda i: (i, 0)),
        out_shape=jax.ShapeDtypeStruct((n // K, co), jnp.float32),
    )(xj, rel, w1a, w1b, b1.reshape(1, -1), w2, b2.reshape(1, -1),
      w3, b3.reshape(1, -1))


def _glob_kernel(feat_ref, w1_ref, b1_ref, w2_ref, b2_ref, w3_ref, b3_ref,
                 wh1_ref, bh1_ref, wh2_ref, bh2_ref, out_ref):
    h = jnp.dot(feat_ref[...], w1_ref[...], preferred_element_type=jnp.float32)
    h = jnp.maximum(h + b1_ref[...], 0.0)
    h = jnp.dot(h, w2_ref[...], preferred_element_type=jnp.float32)
    h = jnp.maximum(h + b2_ref[...], 0.0)
    h = jnp.dot(h, w3_ref[...], preferred_element_type=jnp.float32)
    h = h + b3_ref[...]
    g = jnp.max(h, axis=0, keepdims=True)
    hh = jnp.dot(g, wh1_ref[...], preferred_element_type=jnp.float32)
    hh = jnp.maximum(hh + bh1_ref[...], 0.0)
    logits = jnp.dot(hh, wh2_ref[...], preferred_element_type=jnp.float32)
    logits = logits + bh2_ref[...]
    m = jnp.max(logits, axis=1, keepdims=True)
    e = jnp.exp(logits - m)
    ls = (logits - m) - jnp.log(jnp.sum(e, axis=1, keepdims=True))
    out_ref[...] = ls[None]


def _run_glob(feat, layers, head_layers):
    (w1, b1), (w2, b2), (w3, b3) = layers
    (wh1, bh1), (wh2, bh2) = head_layers
    ci = feat.shape[1]
    full = lambda a: pl.BlockSpec(a.shape, lambda i: (0,) * a.ndim)
    return pl.pallas_call(
        _glob_kernel,
        grid=(B,),
        in_specs=[
            pl.BlockSpec((S2, ci), lambda i: (i, 0)),
            full(w1), full(b1.reshape(1, -1)),
            full(w2), full(b2.reshape(1, -1)),
            full(w3), full(b3.reshape(1, -1)),
            full(wh1), full(bh1.reshape(1, -1)),
            full(wh2), full(bh2.reshape(1, -1)),
        ],
        out_specs=pl.BlockSpec((1, 1, NUM_CLASS), lambda i: (i, 0, 0)),
        out_shape=jax.ShapeDtypeStruct((B, 1, NUM_CLASS), jnp.float32),
    )(feat, w1, b1.reshape(1, -1), w2, b2.reshape(1, -1), w3,
      b3.reshape(1, -1), wh1, bh1.reshape(1, -1), wh2,
      bh2.reshape(1, -1)).reshape(B, NUM_CLASS)


_BUF = K + 2 * SC_L
_NBKT = 16


def _bucket_of(z16):
    b = ((z16 + 1.0) * (_NBKT / 2.0)).astype(jnp.int32)
    return jnp.clip(b, 0, _NBKT - 1)


def _build_zbuckets(pxv, pyv, pzv, ppxv, ppyv, ppzv, ppiv, startsv, n_pts,
                    iota16):
    n_chunks = n_pts // SC_L
    cnt = jnp.int32(0)
    for k in range(_NBKT):
        plsc.store_scatter(startsv, [jnp.full((SC_L,), k, jnp.int32)],
                           jnp.full((SC_L,), cnt, jnp.int32))

        def chunk(c, cnt, k=k):
            z = pzv[pl.ds(c * SC_L, SC_L)]
            mask = _bucket_of(z) == k
            mi = mask.astype(jnp.int32)
            cums = plsc.cumsum(mi)
            slots = jnp.where(mask, cnt + cums - mi, n_pts + iota16)
            plsc.store_scatter(ppiv, [slots], iota16 + c * SC_L)
            return cnt + cums[SC_L - 1]

        cnt = lax.fori_loop(0, n_chunks, chunk, cnt)
    plsc.store_scatter(startsv, [jnp.full((SC_L,), _NBKT, jnp.int32)],
                       jnp.full((SC_L,), n_pts, jnp.int32))

    def fill(c, _):
        idxv = ppiv[pl.ds(c * SC_L, SC_L)]
        ppxv[pl.ds(c * SC_L, SC_L)] = plsc.load_gather(pxv, [idxv])
        ppyv[pl.ds(c * SC_L, SC_L)] = plsc.load_gather(pyv, [idxv])
        ppzv[pl.ds(c * SC_L, SC_L)] = plsc.load_gather(pzv, [idxv])
        return 0

    lax.fori_loop(0, n_chunks, fill, 0)


def _search_row(ppxv, ppyv, ppzv, ppiv, bufv, qxs, qys, qzs, selfs, rr,
                c0, c1, iota16):
    trash = K + SC_L + iota16
    for s in range(_BUF // SC_L):
        bufv[pl.ds(s * SC_L, SC_L)] = selfs

    def chunk(c, cnt):
        base = c * SC_L
        dx = ppxv[pl.ds(base, SC_L)] - qxs
        dy = ppyv[pl.ds(base, SC_L)] - qys
        dz = ppzv[pl.ds(base, SC_L)] - qzs
        dsq = dx * dx + dy * dy + dz * dz
        mask = dsq <= rr
        mi = mask.astype(jnp.int32)
        cums = plsc.cumsum(mi)
        slots = jnp.where(mask, cnt + cums - mi, trash)
        plsc.store_scatter(bufv, [slots], ppiv[pl.ds(base, SC_L)])
        return jnp.minimum(cnt + cums[SC_L - 1], K)

    lax.fori_loop(c0, c1, chunk, 0)


_QW1 = S1 // 2


def _rs1_kernel(px_hbm, py_hbm, pz_hbm, qx_hbm, qy_hbm, qz_hbm, self_hbm,
                rx_hbm, ry_hbm, rz_hbm,
                pxv, pyv, pzv, qxv, qyv, qzv, selfv, bufv, rxv, ryv, rzv,
                ppxv, ppyv, ppzv, ppiv, startsv):
    wid = lax.axis_index("s") * SC_NC + lax.axis_index("c")
    b = wid // 2
    h = wid % 2
    pltpu.sync_copy(px_hbm.at[b], pxv)
    pltpu.sync_copy(py_hbm.at[b], pyv)
    pltpu.sync_copy(pz_hbm.at[b], pzv)
    q0 = h * _QW1
    pltpu.sync_copy(qx_hbm.at[b, pl.ds(q0, _QW1)], qxv)
    pltpu.sync_copy(qy_hbm.at[b, pl.ds(q0, _QW1)], qyv)
    pltpu.sync_copy(qz_hbm.at[b, pl.ds(q0, _QW1)], qzv)
    pltpu.sync_copy(self_hbm.at[b, pl.ds(q0, _QW1)], selfv)
    iota16 = lax.broadcasted_iota(jnp.int32, (SC_L,), 0)
    r = 0.2
    rr = jnp.float32(r * r)
    _build_zbuckets(pxv, pyv, pzv, ppxv, ppyv, ppzv, ppiv, startsv, P,
                    iota16)

    def qchunk(qb, _):
        qx16 = qxv[pl.ds(qb * SC_L, SC_L)]
        qy16 = qyv[pl.ds(qb * SC_L, SC_L)]
        qz16 = qzv[pl.ds(qb * SC_L, SC_L)]
        self16 = selfv[pl.ds(qb * SC_L, SC_L)]
        s16 = plsc.load_gather(startsv, [_bucket_of(qz16 - r)])
        e16 = plsc.load_gather(startsv, [_bucket_of(qz16 + r) + 1])
        for j in range(SC_L):
            qi = qb * SC_L + j
            qxs = jnp.full((SC_L,), qx16[j], jnp.float32)
            qys = jnp.full((SC_L,), qy16[j], jnp.float32)
            qzs = jnp.full((SC_L,), qz16[j], jnp.float32)
            selfs = jnp.full((SC_L,), self16[j], jnp.int32)
            c0 = lax.shift_right_logical(s16[j], 4)
            c1 = lax.shift_right_logical(e16[j] + (SC_L - 1), 4)
            _search_row(ppxv, ppyv, ppzv, ppiv, bufv, qxs, qys, qzs, selfs,
                        rr, c0, c1, iota16)
            for s in range(K // SC_L):
                idxv = bufv[pl.ds(s * SC_L, SC_L)]
                rxv[qi, pl.ds(s * SC_L, SC_L)] = (
                    plsc.load_gather(pxv, [idxv]) - qxs)
                ryv[qi, pl.ds(s * SC_L, SC_L)] = (
                    plsc.load_gather(pyv, [idxv]) - qys)
                rzv[qi, pl.ds(s * SC_L, SC_L)] = (
                    plsc.load_gather(pzv, [idxv]) - qzs)
        return 0

    lax.fori_loop(0, _QW1 // SC_L, qchunk, 0)
    pltpu.sync_copy(rxv, rx_hbm.at[b, pl.ds(q0, _QW1)])
    pltpu.sync_copy(ryv, ry_hbm.at[b, pl.ds(q0, _QW1)])
    pltpu.sync_copy(rzv, rz_hbm.at[b, pl.ds(q0, _QW1)])


def _run_rs1(px, py, pz, qx, qy, qz, self_idx):
    mesh = plsc.VectorSubcoreMesh(core_axis_name="c", subcore_axis_name="s",
                                  num_cores=SC_NC, num_subcores=SC_NS)
    f32 = jnp.float32
    out_type = tuple(jax.ShapeDtypeStruct((B, S1, K), f32) for _ in range(3))
    fn = pl.kernel(
        _rs1_kernel,
        out_type=out_type,
        mesh=mesh,
        scratch_types=[
            pltpu.VMEM((P,), f32), pltpu.VMEM((P,), f32),
            pltpu.VMEM((P,), f32),
            pltpu.VMEM((_QW1,), f32), pltpu.VMEM((_QW1,), f32),
            pltpu.VMEM((_QW1,), f32),
            pltpu.VMEM((_QW1,), jnp.int32),
            pltpu.VMEM((_BUF,), jnp.int32),
            pltpu.VMEM((_QW1, K), f32), pltpu.VMEM((_QW1, K), f32),
            pltpu.VMEM((_QW1, K), f32),
            pltpu.VMEM((P,), f32), pltpu.VMEM((P,), f32),
            pltpu.VMEM((P,), f32),
            pltpu.VMEM((P + SC_L,), jnp.int32),
            pltpu.VMEM((2 * SC_L,), jnp.int32),
        ],
        compiler_params=pltpu.CompilerParams(needs_layout_passes=False),
    )
    return fn(px, py, pz, qx, qy, qz, self_idx)


_QW2 = S2 // 2
_GRP = 4


def _rs2_kernel(px_hbm, py_hbm, pz_hbm, qx_hbm, qy_hbm, qz_hbm, self_hbm,
                x1_hbm,
                rx_hbm, ry_hbm, rz_hbm, xj_hbm,
                pxv, pyv, pzv, qxv, qyv, qzv, selfv, bufv, rxv, ryv, rzv,
                idxg0, idxg1, rows_v0, rows_v1, sem0, sem1,
                ppxv, ppyv, ppzv, ppiv, startsv):
    wid = lax.axis_index("s") * SC_NC + lax.axis_index("c")
    b = wid // 2
    h = wid % 2
    pltpu.sync_copy(px_hbm.at[b], pxv)
    pltpu.sync_copy(py_hbm.at[b], pyv)
    pltpu.sync_copy(pz_hbm.at[b], pzv)
    q0 = h * _QW2
    pltpu.sync_copy(qx_hbm.at[b, pl.ds(q0, _QW2)], qxv)
    pltpu.sync_copy(qy_hbm.at[b, pl.ds(q0, _QW2)], qyv)
    pltpu.sync_copy(qz_hbm.at[b, pl.ds(q0, _QW2)], qzv)
    pltpu.sync_copy(self_hbm.at[b, pl.ds(q0, _QW2)], selfv)
    iota16 = lax.broadcasted_iota(jnp.int32, (SC_L,), 0)
    r = 0.4
    rr = jnp.float32(r * r)
    row_base = jnp.int32(b * S2 + q0)
    _build_zbuckets(pxv, pyv, pzv, ppxv, ppyv, ppzv, ppiv, startsv, S1,
                    iota16)

    idxgs = (idxg0, idxg1)
    rows = (rows_v0, rows_v1)
    sems = (sem0, sem1)
    n_grp = SC_L // _GRP

    def qchunk(qb, _):
        qx16 = qxv[pl.ds(qb * SC_L, SC_L)]
        qy16 = qyv[pl.ds(qb * SC_L, SC_L)]
        qz16 = qzv[pl.ds(qb * SC_L, SC_L)]
        self16 = selfv[pl.ds(qb * SC_L, SC_L)]
        s16 = plsc.load_gather(startsv, [_bucket_of(qz16 - r)])
        e16 = plsc.load_gather(startsv, [_bucket_of(qz16 + r) + 1])
        copies = []
        for gg in range(n_grp):
            idxg = idxgs[gg % 2]
            for j in range(_GRP):
                lane = gg * _GRP + j
                qi = qb * SC_L + lane
                qxs = jnp.full((SC_L,), qx16[lane], jnp.float32)
                qys = jnp.full((SC_L,), qy16[lane], jnp.float32)
                qzs = jnp.full((SC_L,), qz16[lane], jnp.float32)
                selfs = jnp.full((SC_L,), self16[lane], jnp.int32)
                c0 = lax.shift_right_logical(s16[lane], 4)
                c1 = lax.shift_right_logical(e16[lane] + (SC_L - 1), 4)
                _search_row(ppxv, ppyv, ppzv, ppiv, bufv, qxs, qys, qzs,
                            selfs, rr, c0, c1, iota16)
                for s in range(K // SC_L):
                    idxv = bufv[pl.ds(s * SC_L, SC_L)]
                    rxv[qi, pl.ds(s * SC_L, SC_L)] = (
                        plsc.load_gather(pxv, [idxv]) - qxs)
                    ryv[qi, pl.ds(s * SC_L, SC_L)] = (
                        plsc.load_gather(pyv, [idxv]) - qys)
                    rzv[qi, pl.ds(s * SC_L, SC_L)] = (
                        plsc.load_gather(pzv, [idxv]) - qzs)
                    idxg[pl.ds(j * K + s * SC_L, SC_L)] = idxv + b * S1
            copies.append(
                pltpu.async_copy(x1_hbm.at[idxg], rows[gg % 2],
                                 sems[gg % 2]))
            if gg >= 1:
                copies[gg - 1].wait()
                row0 = row_base + qb * SC_L + (gg - 1) * _GRP
                pltpu.sync_copy(rows[(gg - 1) % 2],
                                xj_hbm.at[pl.ds(row0 * K, _GRP * K)])
        copies[n_grp - 1].wait()
        row0 = row_base + qb * SC_L + (n_grp - 1) * _GRP
        pltpu.sync_copy(rows[(n_grp - 1) % 2],
                        xj_hbm.at[pl.ds(row0 * K, _GRP * K)])
        return 0

    lax.fori_loop(0, _QW2 // SC_L, qchunk, 0)
    pltpu.sync_copy(rxv, rx_hbm.at[b, pl.ds(q0, _QW2)])
    pltpu.sync_copy(ryv, ry_hbm.at[b, pl.ds(q0, _QW2)])
    pltpu.sync_copy(rzv, rz_hbm.at[b, pl.ds(q0, _QW2)])


def _run_rs2(px, py, pz, qx, qy, qz, self_idx, x1):
    mesh = plsc.VectorSubcoreMesh(core_axis_name="c", subcore_axis_name="s",
                                  num_cores=SC_NC, num_subcores=SC_NS)
    f32 = jnp.float32
    out_type = (
        jax.ShapeDtypeStruct((B, S2, K), f32),
        jax.ShapeDtypeStruct((B, S2, K), f32),
        jax.ShapeDtypeStruct((B, S2, K), f32),
        jax.ShapeDtypeStruct((B * S2 * K, 128), f32),
    )
    fn = pl.kernel(
        _rs2_kernel,
        out_type=out_type,
        mesh=mesh,
        scratch_types=[
            pltpu.VMEM((S1,), f32), pltpu.VMEM((S1,), f32),
            pltpu.VMEM((S1,), f32),
            pltpu.VMEM((_QW2,), f32), pltpu.VMEM((_QW2,), f32),
            pltpu.VMEM((_QW2,), f32),
            pltpu.VMEM((_QW2,), jnp.int32),
            pltpu.VMEM((_BUF,), jnp.int32),
            pltpu.VMEM((_QW2, K), f32), pltpu.VMEM((_QW2, K), f32),
            pltpu.VMEM((_QW2, K), f32),
            pltpu.VMEM((_GRP * K,), jnp.int32),
            pltpu.VMEM((_GRP * K,), jnp.int32),
            pltpu.VMEM((_GRP * K, 128), f32),
            pltpu.VMEM((_GRP * K, 128), f32),
            pltpu.SemaphoreType.DMA,
            pltpu.SemaphoreType.DMA,
            pltpu.VMEM((S1,), f32), pltpu.VMEM((S1,), f32),
            pltpu.VMEM((S1,), f32),
            pltpu.VMEM((S1 + SC_L,), jnp.int32),
            pltpu.VMEM((2 * SC_L,), jnp.int32),
        ],
        compiler_params=pltpu.CompilerParams(needs_layout_passes=False),
    )
    return fn(px, py, pz, qx, qy, qz, self_idx, x1)


def kernel(pos, batch, params):
    del batch
    pos = pos.reshape(B, P, 3)
    px, py, pz = pos[:, :, 0], pos[:, :, 1], pos[:, :, 2]
    (idx1, p1x, p1y, p1z, idx2, p2x, p2y, p2z) = _run_fps(px, py, pz)

    rx1, ry1, rz1 = _run_rs1(px, py, pz, p1x, p1y, p1z, idx1)
    rel1 = jnp.stack(
        [rx1.reshape(-1), ry1.reshape(-1), rz1.reshape(-1)], axis=-1)
    x1 = _run_pc1(rel1, params['sa1'])

    rx2, ry2, rz2, xj2 = _run_rs2(p1x, p1y, p1z, p2x, p2y, p2z, idx2, x1)
    rel2 = jnp.stack(
        [rx2.reshape(-1), ry2.reshape(-1), rz2.reshape(-1)], axis=-1)
    x2 = _run_pc2(xj2, rel2, params['sa2'])

    pos2 = jnp.stack([p2x, p2y, p2z], axis=-1)
    feat = jnp.concatenate([x2, pos2.reshape(B * S2, 3)], axis=-1)
    return _run_glob(feat, params['sa3'], params['head'])
